# Initial kernel scaffold; baseline (speedup 1.0000x reference)
import jax
import jax.numpy as jnp
from jax.experimental import pallas as pl


def kernel(h, x, edge_feat, edge_index, W_e0, b_e0, W_e1, b_e1, W_inf, b_inf, W_x0, b_x0, W_x1, b_x1, W_n0, b_n0, W_n1, b_n1):
    raise NotImplementedError("write your pallas kernel here")


# speedup vs baseline: 3.7190x; 3.7190x over previous
import functools

import jax
import jax.numpy as jnp
from jax import lax
from jax.experimental import pallas as pl
from jax.experimental.pallas import tpu as pltpu
from jax.experimental.pallas import tpu_sc as plsc

_NC = 2
_NS = 16
_NW = _NC * _NS
_L = 16
_CH = 512
_SUB = 128


def _sc_mesh():
    return plsc.VectorSubcoreMesh(
        core_axis_name="c", subcore_axis_name="s", num_cores=_NC,
        num_subcores=_NS)


def _table_body(h_ref, w_ref, out_ref):
    out_ref[...] = jnp.dot(h_ref[...], w_ref[...],
                           preferred_element_type=jnp.float32)


def _tables_call(h, w_cat, bn):
    n = h.shape[0]
    return pl.pallas_call(
        _table_body,
        grid=(n // bn,),
        in_specs=[
            pl.BlockSpec((bn, h.shape[1]), lambda i: (i, 0)),
            pl.BlockSpec(w_cat.shape, lambda i: (0, 0)),
        ],
        out_specs=pl.BlockSpec((bn, w_cat.shape[1]), lambda i: (i, 0)),
        out_shape=jax.ShapeDtypeStruct((n, w_cat.shape[1]), jnp.float32),
    )(h, w_cat)


def _efhead_call(efp, w_til, br):
    r = efp.shape[0]
    return pl.pallas_call(
        _table_body,
        grid=(r // br,),
        in_specs=[
            pl.BlockSpec((br, 128), lambda i: (i, 0)),
            pl.BlockSpec(w_til.shape, lambda i: (0, 0)),
        ],
        out_specs=pl.BlockSpec((br, w_til.shape[1]), lambda i: (i, 0)),
        out_shape=jax.ShapeDtypeStruct((r, w_til.shape[1]), jnp.float32),
    )(efp, w_til)


def _gather_call(tp, tq, dst3, src3, xflat, e_total):
    n_ch = e_total // _CH
    n_pair = n_ch // 2
    base_p, extra_p = divmod(n_pair, _NW)
    rrows = 8 * n_pair + 8
    pk = jax.ShapeDtypeStruct((rrows, 128), jnp.float32)

    @functools.partial(
        pl.kernel,
        out_type=(
            jax.ShapeDtypeStruct((e_total, 128), jnp.float32),
            jax.ShapeDtypeStruct((e_total, 128), jnp.float32),
            pk, pk, pk, pk,
        ),
        mesh=_sc_mesh(),
        scratch_types=[
            pltpu.VMEM((4, 1, _SUB), jnp.int32),
            pltpu.VMEM((4, 1, _SUB), jnp.int32),
            pltpu.VMEM((_CH, 128), jnp.float32),
            pltpu.VMEM((4 * tp.shape[0],), jnp.float32),
            pltpu.VMEM((8, 128), jnp.float32),
            pltpu.VMEM((8, 128), jnp.float32),
            pltpu.VMEM((8, 128), jnp.float32),
            pltpu.VMEM((8, 128), jnp.float32),
            pltpu.SemaphoreType.DMA,
        ],
        compiler_params=pltpu.CompilerParams(needs_layout_passes=False),
    )
    def k(tp_hbm, tq_hbm, dst_hbm, src_hbm, x_hbm,
          fd_hbm, fs_hbm, rx_hbm, ry_hbm, rz_hbm, r_hbm,
          didx, sidx, rows, xtab, rxb, ryb, rzb, rb, sem):
        cid = lax.axis_index("c")
        sid = lax.axis_index("s")
        wid = sid * _NC + cid
        trips = jnp.where(wid < extra_p, base_p + 1, base_p)

        pltpu.sync_copy(x_hbm, xtab)

        def do_chunk(c, half):
            pltpu.sync_copy(dst_hbm.at[pl.ds(4 * c, 4)], didx)
            pltpu.sync_copy(src_hbm.at[pl.ds(4 * c, 4)], sidx)
            cps = [pltpu.async_copy(tp_hbm.at[didx.at[j, 0]],
                                    rows.at[pl.ds(j * _SUB, _SUB)], sem)
                   for j in range(4)]
            for cp in cps:
                cp.wait()
            pltpu.sync_copy(rows, fd_hbm.at[pl.ds(c * _CH, _CH)])
            cps = [pltpu.async_copy(tq_hbm.at[sidx.at[j, 0]],
                                    rows.at[pl.ds(j * _SUB, _SUB)], sem)
                   for j in range(4)]
            for cp in cps:
                cp.wait()
            pltpu.sync_copy(rows, fs_hbm.at[pl.ds(c * _CH, _CH)])
            for v in range(32):
                row = v // 8
                l0 = (v % 8) * _L
                d = didx[row, 0, pl.ds(l0, _L)]
                s = sidx[row, 0, pl.ds(l0, _L)]
                d4 = d * 4
                s4 = s * 4
                relc = []
                for comp in range(3):
                    xd = plsc.load_gather(xtab, [d4 + comp])
                    xs = plsc.load_gather(xtab, [s4 + comp])
                    relc.append(xd - xs)
                r2 = relc[0] * relc[0] + relc[1] * relc[1] + relc[2] * relc[2]
                prow = 4 * half + row
                rxb[prow, pl.ds(l0, _L)] = relc[0]
                ryb[prow, pl.ds(l0, _L)] = relc[1]
                rzb[prow, pl.ds(l0, _L)] = relc[2]
                rb[prow, pl.ds(l0, _L)] = r2

        def pair(t, _):
            p = wid + t * _NW
            do_chunk(2 * p, 0)
            do_chunk(2 * p + 1, 1)
            pltpu.sync_copy(rxb, rx_hbm.at[pl.ds(8 * p, 8)])
            pltpu.sync_copy(ryb, ry_hbm.at[pl.ds(8 * p, 8)])
            pltpu.sync_copy(rzb, rz_hbm.at[pl.ds(8 * p, 8)])
            pltpu.sync_copy(rb, r_hbm.at[pl.ds(8 * p, 8)])
            return 0

        lax.fori_loop(0, trips, pair, 0, unroll=False)

        @pl.when(wid == 0)
        def _():
            do_chunk(n_ch - 1, 0)
            pltpu.sync_copy(rxb, rx_hbm.at[pl.ds(8 * n_pair, 8)])
            pltpu.sync_copy(ryb, ry_hbm.at[pl.ds(8 * n_pair, 8)])
            pltpu.sync_copy(rzb, rz_hbm.at[pl.ds(8 * n_pair, 8)])
            pltpu.sync_copy(rb, r_hbm.at[pl.ds(8 * n_pair, 8)])

    return k(tp, tq, dst3, src3, xflat)


def _edge_body(ng, r_coeff, r_step, gpb, fd_ref, fs_ref, g0_ref, rp_ref,
               wb_ref, be0_ref, we1_ref, be1_ref, winf_ref, binf_ref,
               wx0_ref, bx0_ref, wx1_ref, bx1_ref, out_ref, xw_ref):
    bc = gpb * 128
    rpk = rp_ref[0]
    rt = jnp.transpose(rpk)
    r = jnp.concatenate([rt[:, g:g + 1] for g in range(gpb)], axis=0)
    offs = lax.broadcasted_iota(jnp.int32, (1, ng), 1).astype(jnp.float32)
    offs = offs * r_step
    r_feat = jnp.exp(r_coeff * (r - offs) ** 2)
    pre = (fd_ref[...] + fs_ref[...] + g0_ref[...] + be0_ref[...]
           + jnp.dot(r_feat, wb_ref[...], preferred_element_type=jnp.float32))
    u = jnp.maximum(pre, 0.0)
    mij = jnp.maximum(
        jnp.dot(u, we1_ref[...], preferred_element_type=jnp.float32)
        + be1_ref[...], 0.0)
    z = jnp.sum(mij * winf_ref[...], axis=-1, keepdims=True) + binf_ref[...]
    eij = 1.0 / (1.0 + jnp.exp(-z))
    t = jnp.maximum(
        jnp.dot(mij, wx0_ref[...], preferred_element_type=jnp.float32)
        + bx0_ref[...], 0.0)
    xw = jnp.sum(t * wx1_ref[...], axis=-1, keepdims=True) + bx1_ref[...]
    out_ref[...] = mij * eij
    xw_ref[...] = jnp.reshape(xw, (1, gpb, 128))


def _edge_call(fd, fs, g0, rp3, wb, b_e0, w_e1, b_e1, w_inf, b_inf,
               w_x0, b_x0, w_x1, b_x1, ng, r_coeff, r_step, bc):
    e_total = fd.shape[0]
    gpb = bc // 128
    nb = e_total // bc
    full = lambda a: pl.BlockSpec(a.shape, lambda i: tuple(0 for _ in a.shape))
    return pl.pallas_call(
        functools.partial(_edge_body, ng, r_coeff, r_step, gpb),
        grid=(nb,),
        in_specs=[
            pl.BlockSpec((bc, 128), lambda i: (i, 0)),
            pl.BlockSpec((bc, 128), lambda i: (i, 0)),
            pl.BlockSpec((bc, 128), lambda i: (i, 0)),
            pl.BlockSpec((1, gpb, 128), lambda i: (i, 0, 0)),
            full(wb), full(b_e0), full(w_e1), full(b_e1), full(w_inf),
            full(b_inf), full(w_x0), full(b_x0), full(w_x1), full(b_x1),
        ],
        out_specs=[
            pl.BlockSpec((bc, 128), lambda i: (i, 0)),
            pl.BlockSpec((1, gpb, 128), lambda i: (i, 0, 0)),
        ],
        out_shape=[
            jax.ShapeDtypeStruct((e_total, 128), jnp.float32),
            jax.ShapeDtypeStruct((nb, gpb, 128), jnp.float32),
        ],
    )(fd, fs, g0, rp3, wb, b_e0, w_e1, b_e1, w_inf, b_inf, w_x0, b_x0,
      w_x1, b_x1)


def _scatter_call(gated, dst3, xwp, rxp, ryp, rzp, zeros_nf, n_total,
                  e_total):
    n_ch = e_total // _CH
    n_pair = n_ch // 2
    base_p, extra_p = divmod(n_pair, _NS)
    n_half = n_total // _NC
    rpt = 312
    xacc_len = 1280 * _NS

    @functools.partial(
        pl.kernel,
        out_type=(
            jax.ShapeDtypeStruct((_NC, n_half, 128), jnp.float32),
            jax.ShapeDtypeStruct((_NC, xacc_len), jnp.float32),
        ),
        mesh=_sc_mesh(),
        scratch_types=[
            pltpu.VMEM((4, 1, _SUB), jnp.int32),
            pltpu.VMEM((4, 1, _SUB), jnp.int32),
            pltpu.VMEM((_CH, 128), jnp.float32),
            pltpu.VMEM((8, 128), jnp.float32),
            pltpu.VMEM((8, 128), jnp.float32),
            pltpu.VMEM((8, 128), jnp.float32),
            pltpu.VMEM((8, 128), jnp.float32),
            pltpu.VMEM((2048,), jnp.float32),
            pltpu.VMEM((16, 1, _SUB), jnp.int32),
            pltpu.VMEM((1280,), jnp.float32),
            pltpu.VMEM_SHARED((n_half, 128), jnp.float32),
            pltpu.VMEM_SHARED((xacc_len,), jnp.float32),
            pltpu.SemaphoreType.DMA,
        ],
        compiler_params=pltpu.CompilerParams(needs_layout_passes=False),
    )
    def k(g_hbm, dst_hbm, xw_hbm, rx_hbm, ry_hbm, rz_hbm, z_hbm,
          acc_hbm, xacc_hbm,
          didx, fidx, grow, xwb, rxb, ryb, rzb, vals, idxs, zbuf, acc, xacc,
          sem):
        cid = lax.axis_index("c")
        sid = lax.axis_index("s")
        nbase = cid * n_half
        trips = jnp.where(sid < extra_p, base_p + 1, base_p)

        def zloop(i, _):
            zbuf[pl.ds(i * _L, _L)] = jnp.zeros((_L,), jnp.float32)
            return 0
        lax.fori_loop(0, 1280 // _L, zloop, 0, unroll=False)
        pltpu.sync_copy(zbuf, xacc.at[pl.ds(sid * 1280, 1280)])

        @pl.when(sid < _NS - 1)
        def _():
            pltpu.sync_copy(z_hbm.at[pl.ds(0, rpt)],
                            acc.at[pl.ds(sid * rpt, rpt)])

        @pl.when(sid == _NS - 1)
        def _():
            pltpu.sync_copy(z_hbm.at[pl.ds(0, 320)],
                            acc.at[pl.ds((_NS - 1) * rpt, 320)])

        plsc.subcore_barrier()

        def do_chunk(c, half):
            pltpu.sync_copy(dst_hbm.at[pl.ds(4 * c, 4)], didx)
            cp = pltpu.async_copy(g_hbm.at[pl.ds(c * _CH, _CH)], grow, sem)
            cp.wait()
            for v in range(32):
                row = v // 8
                l0 = (v % 8) * _L
                prow = 4 * half + row
                d = didx[row, 0, pl.ds(l0, _L)] - nbase
                valid = (d >= 0) & (d < n_half)
                fidx[row, 0, pl.ds(l0, _L)] = jnp.where(valid, d, -1)
                xw = xwb[prow, pl.ds(l0, _L)]
                base = v * 64
                pos0 = lax.iota(jnp.int32, _L) * 4
                for comp, rbuf in ((0, rxb), (1, ryb), (2, rzb)):
                    val = rbuf[prow, pl.ds(l0, _L)] * xw
                    pos = pos0 + (base + comp)
                    plsc.store_scatter(vals, [pos], val)
                    plsc.store_scatter(
                        idxs, [pos // _SUB,
                               jnp.zeros((_L,), jnp.int32),
                               lax.rem(pos, _SUB)],
                        jnp.where(valid, d * 4 + comp, -1))
                pos = pos0 + (base + 3)
                plsc.store_scatter(
                    idxs, [pos // _SUB,
                           jnp.zeros((_L,), jnp.int32),
                           lax.rem(pos, _SUB)],
                    jnp.full((_L,), -1, jnp.int32))
            for j in range(4):
                pltpu.sync_copy(
                    grow.at[pl.ds(j * _SUB, _SUB)],
                    acc.at[plsc.Indices(fidx.at[j, 0], ignored_value=-1)],
                    add=True)
            for s in range(16):
                pltpu.sync_copy(
                    vals.at[pl.ds(s * _SUB, _SUB)],
                    xacc.at[plsc.Indices(idxs.at[s, 0], ignored_value=-1)],
                    add=True)

        def pair(t, _):
            p = sid + t * _NS
            pltpu.sync_copy(xw_hbm.at[pl.ds(8 * p, 8)], xwb)
            pltpu.sync_copy(rx_hbm.at[pl.ds(8 * p, 8)], rxb)
            pltpu.sync_copy(ry_hbm.at[pl.ds(8 * p, 8)], ryb)
            pltpu.sync_copy(rz_hbm.at[pl.ds(8 * p, 8)], rzb)
            do_chunk(2 * p, 0)
            do_chunk(2 * p + 1, 1)
            return 0

        lax.fori_loop(0, trips, pair, 0, unroll=False)

        @pl.when(sid == 0)
        def _():
            pltpu.sync_copy(xw_hbm.at[pl.ds(8 * n_pair, 8)], xwb)
            pltpu.sync_copy(rx_hbm.at[pl.ds(8 * n_pair, 8)], rxb)
            pltpu.sync_copy(ry_hbm.at[pl.ds(8 * n_pair, 8)], ryb)
            pltpu.sync_copy(rz_hbm.at[pl.ds(8 * n_pair, 8)], rzb)
            do_chunk(n_ch - 1, 0)

        plsc.subcore_barrier()

        @pl.when(sid < _NS - 1)
        def _():
            pltpu.sync_copy(acc.at[pl.ds(sid * rpt, rpt)],
                            acc_hbm.at[cid, pl.ds(sid * rpt, rpt)])

        @pl.when(sid == _NS - 1)
        def _():
            pltpu.sync_copy(acc.at[pl.ds((_NS - 1) * rpt, 320)],
                            acc_hbm.at[cid, pl.ds((_NS - 1) * rpt, 320)])

        pltpu.sync_copy(xacc.at[pl.ds(sid * 1280, 1280)],
                        xacc_hbm.at[cid, pl.ds(sid * 1280, 1280)])

    return k(gated, dst3, xwp, rxp, ryp, rzp, zeros_nf)


def _node_body(mi_ref, h_ref, wn0a_ref, wn0b_ref, bn0_ref,
               wn1_ref, bn1_ref, hout_ref):
    mi = mi_ref[...]
    h = h_ref[...]
    u = jnp.maximum(
        jnp.dot(mi, wn0a_ref[...], preferred_element_type=jnp.float32)
        + jnp.dot(h, wn0b_ref[...], preferred_element_type=jnp.float32)
        + bn0_ref[...], 0.0)
    hout_ref[...] = h + jnp.dot(
        u, wn1_ref[...], preferred_element_type=jnp.float32) + bn1_ref[...]


def _node_call(mi, h, wn0a, wn0b, b_n0, w_n1, b_n1, bn):
    n = h.shape[0]
    full = lambda a: pl.BlockSpec(a.shape, lambda i: tuple(0 for _ in a.shape))
    return pl.pallas_call(
        _node_body,
        grid=(n // bn,),
        in_specs=[
            pl.BlockSpec((bn, 128), lambda i: (i, 0)),
            pl.BlockSpec((bn, 128), lambda i: (i, 0)),
            full(wn0a), full(wn0b), full(b_n0), full(w_n1), full(b_n1),
        ],
        out_specs=pl.BlockSpec((bn, 128), lambda i: (i, 0)),
        out_shape=jax.ShapeDtypeStruct((n, 128), jnp.float32),
    )(mi, h, wn0a, wn0b, b_n0, w_n1, b_n1)


def _xfin_body(xp_ref, a0_ref, out_ref):
    out_ref[...] = xp_ref[...] + a0_ref[...]


def _xfin_call(xp, a0):
    spec = pl.BlockSpec(xp.shape, lambda: (0, 0))
    return pl.pallas_call(
        _xfin_body,
        in_specs=[spec, spec],
        out_specs=spec,
        out_shape=jax.ShapeDtypeStruct(xp.shape, jnp.float32),
    )(xp, a0)


def kernel(h, x, edge_feat, edge_index, W_e0, b_e0, W_e1, b_e1, W_inf, b_inf,
           W_x0, b_x0, W_x1, b_x1, W_n0, b_n0, W_n1, b_n1):
    n, hd = h.shape
    e = edge_index.shape[1]
    ef = edge_feat.shape[1]
    ng = W_e0.shape[0] - 2 * hd - ef
    r_step = 100.0 / (ng - 1)
    r_coeff = -0.5 / r_step ** 2
    bc = 2560
    gpb = bc // 128

    src = edge_index[0]
    dst = edge_index[1]
    dst3 = dst.reshape(e // _SUB, 1, _SUB)
    src3 = src.reshape(e // _SUB, 1, _SUB)

    w_a = W_e0[0:ef]
    w_b = W_e0[ef:ef + ng]
    w_cat = jnp.concatenate([W_e0[ef + ng:ef + ng + hd],
                             W_e0[ef + ng + hd:]], axis=1)
    w_til = jnp.kron(jnp.eye(8, dtype=jnp.float32), w_a)
    wn0a = W_n0[0:hd]
    wn0b = W_n0[hd:]

    pq = _tables_call(h, w_cat, 1000)
    tp = pq[:, 0:hd]
    tq = pq[:, hd:]
    xflat = jnp.pad(x, ((0, 0), (0, 1))).reshape(-1)

    efp = edge_feat.reshape(e // 8, 128)
    g0 = _efhead_call(efp, w_til, 1000).reshape(e, 128)

    fd, fs, rxp, ryp, rzp, rp = _gather_call(tp, tq, dst3, src3, xflat, e)
    rp3 = rp[0:e // 128].reshape(e // bc, gpb, 128)

    gated, xw3 = _edge_call(
        fd, fs, g0, rp3, w_b, b_e0.reshape(1, -1), W_e1, b_e1.reshape(1, -1),
        W_inf.reshape(1, -1), b_inf.reshape(1, 1), W_x0, b_x0.reshape(1, -1),
        W_x1.reshape(1, -1), b_x1.reshape(1, 1), ng, r_coeff, r_step, bc)

    xwp = jnp.pad(xw3.reshape(e // 128, 128), ((0, rxp.shape[0] - e // 128),
                                               (0, 0)))

    zeros_nf = jnp.zeros((n, 128), jnp.float32)
    acc, xacc = _scatter_call(gated, dst3, xwp, rxp, ryp, rzp, zeros_nf, n, e)

    mi = jnp.concatenate([acc[0], acc[1]], axis=0)
    h_new = _node_call(mi, h, wn0a, wn0b, b_n0.reshape(1, -1),
                       W_n1, b_n1.reshape(1, -1), 1000)

    n_half = n // _NC
    dxf = jnp.concatenate([xacc[0, 0:4 * n_half], xacc[1, 0:4 * n_half]])
    xa = jnp.pad(dxf, (0, 960)).reshape(-1, 128)
    xpad = jnp.pad(x, ((0, 240), (0, 1))).reshape(-1, 128)
    xnp = _xfin_call(xpad, xa)
    x_new = xnp.reshape(-1, 4)[0:n, 0:3]
    return (h_new, x_new)


# speedup vs baseline: 3.7211x; 1.0006x over previous
import functools

import jax
import jax.numpy as jnp
from jax import lax
from jax.experimental import pallas as pl
from jax.experimental.pallas import tpu as pltpu
from jax.experimental.pallas import tpu_sc as plsc

_NC = 2
_NS = 16
_NW = _NC * _NS
_L = 16
_CH = 512
_SUB = 128


def _sc_mesh():
    return plsc.VectorSubcoreMesh(
        core_axis_name="c", subcore_axis_name="s", num_cores=_NC,
        num_subcores=_NS)


def _table_body(h_ref, w_ref, out_ref):
    out_ref[...] = jnp.dot(h_ref[...], w_ref[...],
                           preferred_element_type=jnp.float32)


def _tables_call(h, w_cat, bn):
    n = h.shape[0]
    return pl.pallas_call(
        _table_body,
        grid=(n // bn,),
        in_specs=[
            pl.BlockSpec((bn, h.shape[1]), lambda i: (i, 0)),
            pl.BlockSpec(w_cat.shape, lambda i: (0, 0)),
        ],
        out_specs=pl.BlockSpec((bn, w_cat.shape[1]), lambda i: (i, 0)),
        out_shape=jax.ShapeDtypeStruct((n, w_cat.shape[1]), jnp.float32),
    )(h, w_cat)


def _efhead_call(efp, w_til, br):
    r = efp.shape[0]
    return pl.pallas_call(
        _table_body,
        grid=(r // br,),
        in_specs=[
            pl.BlockSpec((br, 128), lambda i: (i, 0)),
            pl.BlockSpec(w_til.shape, lambda i: (0, 0)),
        ],
        out_specs=pl.BlockSpec((br, w_til.shape[1]), lambda i: (i, 0)),
        out_shape=jax.ShapeDtypeStruct((r, w_til.shape[1]), jnp.float32),
    )(efp, w_til)


def _gather_call(tp, tq, dst3, src3, xflat, e_total):
    grp = 1024
    n_grp = e_total // grp
    base_p, extra_p = divmod(n_grp, _NW)
    rrows = 8 * n_grp + 8
    pk = jax.ShapeDtypeStruct((rrows, 128), jnp.float32)

    @functools.partial(
        pl.kernel,
        out_type=(
            jax.ShapeDtypeStruct((e_total, 128), jnp.float32),
            jax.ShapeDtypeStruct((e_total, 128), jnp.float32),
            pk, pk, pk, pk,
        ),
        mesh=_sc_mesh(),
        scratch_types=[
            pltpu.VMEM((8, 1, _SUB), jnp.int32),
            pltpu.VMEM((8, 1, _SUB), jnp.int32),
            pltpu.VMEM((256, 128), jnp.float32),
            pltpu.VMEM((256, 128), jnp.float32),
            pltpu.VMEM((4 * tp.shape[0],), jnp.float32),
            pltpu.VMEM((8, 128), jnp.float32),
            pltpu.VMEM((8, 128), jnp.float32),
            pltpu.VMEM((8, 128), jnp.float32),
            pltpu.VMEM((8, 128), jnp.float32),
            pltpu.SemaphoreType.DMA,
            pltpu.SemaphoreType.DMA,
            pltpu.SemaphoreType.DMA,
            pltpu.SemaphoreType.DMA,
        ],
        compiler_params=pltpu.CompilerParams(needs_layout_passes=False),
    )
    def k(tp_hbm, tq_hbm, dst_hbm, src_hbm, x_hbm,
          fd_hbm, fs_hbm, rx_hbm, ry_hbm, rz_hbm, r_hbm,
          didx, sidx, bufa, bufb, xtab, rxb, ryb, rzb, rb,
          gs0, gs1, ws0, ws1):
        cid = lax.axis_index("c")
        sid = lax.axis_index("s")
        wid = sid * _NC + cid
        trips = jnp.where(wid < extra_p, base_p + 1, base_p)

        pltpu.sync_copy(x_hbm, xtab)
        bufs = (bufa, bufb)
        gsems = (gs0, gs1)
        wsems = (ws0, ws1)

        def fire_gather(g, s):
            q, is_q = divmod(s, 2)
            tab = tq_hbm if is_q else tp_hbm
            idx = sidx if is_q else didx
            buf = bufs[s % 2]
            sem = gsems[s % 2]
            return [pltpu.async_copy(tab.at[idx.at[2 * q + j, 0]],
                                     buf.at[pl.ds(j * _SUB, _SUB)], sem)
                    for j in range(2)]

        def fire_write(g, s):
            q, is_q = divmod(s, 2)
            out = fs_hbm if is_q else fd_hbm
            return pltpu.async_copy(
                bufs[s % 2], out.at[pl.ds(g * grp + q * 256, 256)],
                wsems[s % 2])

        def rel_compute(q):
            for v in range(16):
                row = 2 * q + v // 8
                l0 = (v % 8) * _L
                d = didx[row, 0, pl.ds(l0, _L)]
                s = sidx[row, 0, pl.ds(l0, _L)]
                relc = []
                for comp in range(3):
                    xd = plsc.load_gather(xtab, [d * 4 + comp])
                    xs = plsc.load_gather(xtab, [s * 4 + comp])
                    relc.append(xd - xs)
                r2 = relc[0] * relc[0] + relc[1] * relc[1] + relc[2] * relc[2]
                rxb[row, pl.ds(l0, _L)] = relc[0]
                ryb[row, pl.ds(l0, _L)] = relc[1]
                rzb[row, pl.ds(l0, _L)] = relc[2]
                rb[row, pl.ds(l0, _L)] = r2

        def do_group(g, n_steps):
            pltpu.sync_copy(dst_hbm.at[pl.ds(8 * g, 8)], didx)
            pltpu.sync_copy(src_hbm.at[pl.ds(8 * g, 8)], sidx)
            gcps = {0: fire_gather(g, 0)}
            wcps = {}
            for s in range(n_steps):
                if s + 1 < n_steps:
                    if s - 1 >= 0:
                        wcps.pop(s - 1).wait()
                    gcps[s + 1] = fire_gather(g, s + 1)
                for cp in gcps.pop(s):
                    cp.wait()
                wcps[s] = fire_write(g, s)
                if s % 2 == 1:
                    rel_compute(s // 2)
            for s in sorted(wcps):
                wcps[s].wait()
            pltpu.sync_copy(rxb, rx_hbm.at[pl.ds(8 * g, 8)])
            pltpu.sync_copy(ryb, ry_hbm.at[pl.ds(8 * g, 8)])
            pltpu.sync_copy(rzb, rz_hbm.at[pl.ds(8 * g, 8)])
            pltpu.sync_copy(rb, r_hbm.at[pl.ds(8 * g, 8)])

        def group(t, _):
            do_group(wid + t * _NW, 8)
            return 0

        lax.fori_loop(0, trips, group, 0, unroll=False)

        @pl.when(wid == 0)
        def _():
            pltpu.sync_copy(dst_hbm.at[pl.ds(8 * n_grp, 4)],
                            didx.at[pl.ds(0, 4)])
            pltpu.sync_copy(src_hbm.at[pl.ds(8 * n_grp, 4)],
                            sidx.at[pl.ds(0, 4)])
            gcps = {0: fire_gather(n_grp, 0)}
            wcps = {}
            for s in range(4):
                if s + 1 < 4:
                    if s - 1 >= 0:
                        wcps.pop(s - 1).wait()
                    gcps[s + 1] = fire_gather(n_grp, s + 1)
                for cp in gcps.pop(s):
                    cp.wait()
                wcps[s] = fire_write(n_grp, s)
                if s % 2 == 1:
                    rel_compute(s // 2)
            for s in sorted(wcps):
                wcps[s].wait()
            pltpu.sync_copy(rxb, rx_hbm.at[pl.ds(8 * n_grp, 8)])
            pltpu.sync_copy(ryb, ry_hbm.at[pl.ds(8 * n_grp, 8)])
            pltpu.sync_copy(rzb, rz_hbm.at[pl.ds(8 * n_grp, 8)])
            pltpu.sync_copy(rb, r_hbm.at[pl.ds(8 * n_grp, 8)])

    return k(tp, tq, dst3, src3, xflat)


def _edge_body(ng, r_coeff, r_step, gpb, fd_ref, fs_ref, g0_ref, rp_ref,
               wb_ref, be0_ref, we1_ref, be1_ref, winf_ref, binf_ref,
               wx0_ref, bx0_ref, wx1_ref, bx1_ref, out_ref, xw_ref):
    bc = gpb * 128
    rpk = rp_ref[0]
    rt = jnp.transpose(rpk)
    r = jnp.concatenate([rt[:, g:g + 1] for g in range(gpb)], axis=0)
    offs = lax.broadcasted_iota(jnp.int32, (1, ng), 1).astype(jnp.float32)
    offs = offs * r_step
    r_feat = jnp.exp(r_coeff * (r - offs) ** 2)
    pre = (fd_ref[...] + fs_ref[...] + g0_ref[...] + be0_ref[...]
           + jnp.dot(r_feat, wb_ref[...], preferred_element_type=jnp.float32))
    u = jnp.maximum(pre, 0.0)
    mij = jnp.maximum(
        jnp.dot(u, we1_ref[...], preferred_element_type=jnp.float32)
        + be1_ref[...], 0.0)
    z = jnp.sum(mij * winf_ref[...], axis=-1, keepdims=True) + binf_ref[...]
    eij = 1.0 / (1.0 + jnp.exp(-z))
    t = jnp.maximum(
        jnp.dot(mij, wx0_ref[...], preferred_element_type=jnp.float32)
        + bx0_ref[...], 0.0)
    xw = jnp.sum(t * wx1_ref[...], axis=-1, keepdims=True) + bx1_ref[...]
    out_ref[...] = mij * eij
    xw_ref[...] = jnp.reshape(xw, (1, gpb, 128))


def _edge_call(fd, fs, g0, rp3, wb, b_e0, w_e1, b_e1, w_inf, b_inf,
               w_x0, b_x0, w_x1, b_x1, ng, r_coeff, r_step, bc):
    e_total = fd.shape[0]
    gpb = bc // 128
    nb = e_total // bc
    full = lambda a: pl.BlockSpec(a.shape, lambda i: tuple(0 for _ in a.shape))
    return pl.pallas_call(
        functools.partial(_edge_body, ng, r_coeff, r_step, gpb),
        grid=(nb,),
        in_specs=[
            pl.BlockSpec((bc, 128), lambda i: (i, 0)),
            pl.BlockSpec((bc, 128), lambda i: (i, 0)),
            pl.BlockSpec((bc, 128), lambda i: (i, 0)),
            pl.BlockSpec((1, gpb, 128), lambda i: (i, 0, 0)),
            full(wb), full(b_e0), full(w_e1), full(b_e1), full(w_inf),
            full(b_inf), full(w_x0), full(b_x0), full(w_x1), full(b_x1),
        ],
        out_specs=[
            pl.BlockSpec((bc, 128), lambda i: (i, 0)),
            pl.BlockSpec((1, gpb, 128), lambda i: (i, 0, 0)),
        ],
        out_shape=[
            jax.ShapeDtypeStruct((e_total, 128), jnp.float32),
            jax.ShapeDtypeStruct((nb, gpb, 128), jnp.float32),
        ],
    )(fd, fs, g0, rp3, wb, b_e0, w_e1, b_e1, w_inf, b_inf, w_x0, b_x0,
      w_x1, b_x1)


def _scatter_call(gated, dst3, xwp, rxp, ryp, rzp, zeros_nf, n_total,
                  e_total):
    n_ch = e_total // _CH
    n_pair = n_ch // 2
    base_p, extra_p = divmod(n_pair, _NS)
    n_half = n_total // _NC
    rpt = 312
    xacc_len = 1280 * _NS

    @functools.partial(
        pl.kernel,
        out_type=(
            jax.ShapeDtypeStruct((_NC, n_half, 128), jnp.float32),
            jax.ShapeDtypeStruct((_NC, xacc_len), jnp.float32),
        ),
        mesh=_sc_mesh(),
        scratch_types=[
            pltpu.VMEM((4, 1, _SUB), jnp.int32),
            pltpu.VMEM((4, 1, _SUB), jnp.int32),
            pltpu.VMEM((_CH, 128), jnp.float32),
            pltpu.VMEM((8, 128), jnp.float32),
            pltpu.VMEM((8, 128), jnp.float32),
            pltpu.VMEM((8, 128), jnp.float32),
            pltpu.VMEM((8, 128), jnp.float32),
            pltpu.VMEM((2048,), jnp.float32),
            pltpu.VMEM((16, 1, _SUB), jnp.int32),
            pltpu.VMEM((1280,), jnp.float32),
            pltpu.VMEM_SHARED((n_half, 128), jnp.float32),
            pltpu.VMEM_SHARED((xacc_len,), jnp.float32),
            pltpu.SemaphoreType.DMA,
        ],
        compiler_params=pltpu.CompilerParams(needs_layout_passes=False),
    )
    def k(g_hbm, dst_hbm, xw_hbm, rx_hbm, ry_hbm, rz_hbm, z_hbm,
          acc_hbm, xacc_hbm,
          didx, fidx, grow, xwb, rxb, ryb, rzb, vals, idxs, zbuf, acc, xacc,
          sem):
        cid = lax.axis_index("c")
        sid = lax.axis_index("s")
        nbase = cid * n_half
        trips = jnp.where(sid < extra_p, base_p + 1, base_p)

        def zloop(i, _):
            zbuf[pl.ds(i * _L, _L)] = jnp.zeros((_L,), jnp.float32)
            return 0
        lax.fori_loop(0, 1280 // _L, zloop, 0, unroll=False)
        pltpu.sync_copy(zbuf, xacc.at[pl.ds(sid * 1280, 1280)])

        @pl.when(sid < _NS - 1)
        def _():
            pltpu.sync_copy(z_hbm.at[pl.ds(0, rpt)],
                            acc.at[pl.ds(sid * rpt, rpt)])

        @pl.when(sid == _NS - 1)
        def _():
            pltpu.sync_copy(z_hbm.at[pl.ds(0, 320)],
                            acc.at[pl.ds((_NS - 1) * rpt, 320)])

        plsc.subcore_barrier()

        def do_chunk(c, half):
            pltpu.sync_copy(dst_hbm.at[pl.ds(4 * c, 4)], didx)
            cp = pltpu.async_copy(g_hbm.at[pl.ds(c * _CH, _CH)], grow, sem)
            cp.wait()
            for v in range(32):
                row = v // 8
                l0 = (v % 8) * _L
                prow = 4 * half + row
                d = didx[row, 0, pl.ds(l0, _L)] - nbase
                valid = (d >= 0) & (d < n_half)
                fidx[row, 0, pl.ds(l0, _L)] = jnp.where(valid, d, -1)
                xw = xwb[prow, pl.ds(l0, _L)]
                base = v * 64
                pos0 = lax.iota(jnp.int32, _L) * 4
                for comp, rbuf in ((0, rxb), (1, ryb), (2, rzb)):
                    val = rbuf[prow, pl.ds(l0, _L)] * xw
                    pos = pos0 + (base + comp)
                    plsc.store_scatter(vals, [pos], val)
                    plsc.store_scatter(
                        idxs, [pos // _SUB,
                               jnp.zeros((_L,), jnp.int32),
                               lax.rem(pos, _SUB)],
                        jnp.where(valid, d * 4 + comp, -1))
                pos = pos0 + (base + 3)
                plsc.store_scatter(
                    idxs, [pos // _SUB,
                           jnp.zeros((_L,), jnp.int32),
                           lax.rem(pos, _SUB)],
                    jnp.full((_L,), -1, jnp.int32))
            for j in range(4):
                pltpu.sync_copy(
                    grow.at[pl.ds(j * _SUB, _SUB)],
                    acc.at[plsc.Indices(fidx.at[j, 0], ignored_value=-1)],
                    add=True)
            for s in range(16):
                pltpu.sync_copy(
                    vals.at[pl.ds(s * _SUB, _SUB)],
                    xacc.at[plsc.Indices(idxs.at[s, 0], ignored_value=-1)],
                    add=True)

        def pair(t, _):
            p = sid + t * _NS
            pltpu.sync_copy(xw_hbm.at[pl.ds(8 * p, 8)], xwb)
            pltpu.sync_copy(rx_hbm.at[pl.ds(8 * p, 8)], rxb)
            pltpu.sync_copy(ry_hbm.at[pl.ds(8 * p, 8)], ryb)
            pltpu.sync_copy(rz_hbm.at[pl.ds(8 * p, 8)], rzb)
            do_chunk(2 * p, 0)
            do_chunk(2 * p + 1, 1)
            return 0

        lax.fori_loop(0, trips, pair, 0, unroll=False)

        @pl.when(sid == 0)
        def _():
            pltpu.sync_copy(xw_hbm.at[pl.ds(8 * n_pair, 8)], xwb)
            pltpu.sync_copy(rx_hbm.at[pl.ds(8 * n_pair, 8)], rxb)
            pltpu.sync_copy(ry_hbm.at[pl.ds(8 * n_pair, 8)], ryb)
            pltpu.sync_copy(rz_hbm.at[pl.ds(8 * n_pair, 8)], rzb)
            do_chunk(n_ch - 1, 0)

        plsc.subcore_barrier()

        @pl.when(sid < _NS - 1)
        def _():
            pltpu.sync_copy(acc.at[pl.ds(sid * rpt, rpt)],
                            acc_hbm.at[cid, pl.ds(sid * rpt, rpt)])

        @pl.when(sid == _NS - 1)
        def _():
            pltpu.sync_copy(acc.at[pl.ds((_NS - 1) * rpt, 320)],
                            acc_hbm.at[cid, pl.ds((_NS - 1) * rpt, 320)])

        pltpu.sync_copy(xacc.at[pl.ds(sid * 1280, 1280)],
                        xacc_hbm.at[cid, pl.ds(sid * 1280, 1280)])

    return k(gated, dst3, xwp, rxp, ryp, rzp, zeros_nf)


def _node_body(mi_ref, h_ref, wn0a_ref, wn0b_ref, bn0_ref,
               wn1_ref, bn1_ref, hout_ref):
    mi = mi_ref[...]
    h = h_ref[...]
    u = jnp.maximum(
        jnp.dot(mi, wn0a_ref[...], preferred_element_type=jnp.float32)
        + jnp.dot(h, wn0b_ref[...], preferred_element_type=jnp.float32)
        + bn0_ref[...], 0.0)
    hout_ref[...] = h + jnp.dot(
        u, wn1_ref[...], preferred_element_type=jnp.float32) + bn1_ref[...]


def _node_call(mi, h, wn0a, wn0b, b_n0, w_n1, b_n1, bn):
    n = h.shape[0]
    full = lambda a: pl.BlockSpec(a.shape, lambda i: tuple(0 for _ in a.shape))
    return pl.pallas_call(
        _node_body,
        grid=(n // bn,),
        in_specs=[
            pl.BlockSpec((bn, 128), lambda i: (i, 0)),
            pl.BlockSpec((bn, 128), lambda i: (i, 0)),
            full(wn0a), full(wn0b), full(b_n0), full(w_n1), full(b_n1),
        ],
        out_specs=pl.BlockSpec((bn, 128), lambda i: (i, 0)),
        out_shape=jax.ShapeDtypeStruct((n, 128), jnp.float32),
    )(mi, h, wn0a, wn0b, b_n0, w_n1, b_n1)


def _xfin_body(xp_ref, a0_ref, out_ref):
    out_ref[...] = xp_ref[...] + a0_ref[...]


def _xfin_call(xp, a0):
    spec = pl.BlockSpec(xp.shape, lambda: (0, 0))
    return pl.pallas_call(
        _xfin_body,
        in_specs=[spec, spec],
        out_specs=spec,
        out_shape=jax.ShapeDtypeStruct(xp.shape, jnp.float32),
    )(xp, a0)


def kernel(h, x, edge_feat, edge_index, W_e0, b_e0, W_e1, b_e1, W_inf, b_inf,
           W_x0, b_x0, W_x1, b_x1, W_n0, b_n0, W_n1, b_n1):
    n, hd = h.shape
    e = edge_index.shape[1]
    ef = edge_feat.shape[1]
    ng = W_e0.shape[0] - 2 * hd - ef
    r_step = 100.0 / (ng - 1)
    r_coeff = -0.5 / r_step ** 2
    bc = 2560
    gpb = bc // 128

    src = edge_index[0]
    dst = edge_index[1]
    dst3 = dst.reshape(e // _SUB, 1, _SUB)
    src3 = src.reshape(e // _SUB, 1, _SUB)

    w_a = W_e0[0:ef]
    w_b = W_e0[ef:ef + ng]
    w_cat = jnp.concatenate([W_e0[ef + ng:ef + ng + hd],
                             W_e0[ef + ng + hd:]], axis=1)
    w_til = jnp.kron(jnp.eye(8, dtype=jnp.float32), w_a)
    wn0a = W_n0[0:hd]
    wn0b = W_n0[hd:]

    pq = _tables_call(h, w_cat, 1000)
    tp = pq[:, 0:hd]
    tq = pq[:, hd:]
    xflat = jnp.pad(x, ((0, 0), (0, 1))).reshape(-1)

    efp = edge_feat.reshape(e // 8, 128)
    g0 = _efhead_call(efp, w_til, 1000).reshape(e, 128)

    fd, fs, rxp, ryp, rzp, rp = _gather_call(tp, tq, dst3, src3, xflat, e)
    rp3 = rp[0:e // 128].reshape(e // bc, gpb, 128)

    gated, xw3 = _edge_call(
        fd, fs, g0, rp3, w_b, b_e0.reshape(1, -1), W_e1, b_e1.reshape(1, -1),
        W_inf.reshape(1, -1), b_inf.reshape(1, 1), W_x0, b_x0.reshape(1, -1),
        W_x1.reshape(1, -1), b_x1.reshape(1, 1), ng, r_coeff, r_step, bc)

    xwp = jnp.pad(xw3.reshape(e // 128, 128), ((0, rxp.shape[0] - e // 128),
                                               (0, 0)))

    zeros_nf = jnp.zeros((n, 128), jnp.float32)
    acc, xacc = _scatter_call(gated, dst3, xwp, rxp, ryp, rzp, zeros_nf, n, e)

    mi = jnp.concatenate([acc[0], acc[1]], axis=0)
    h_new = _node_call(mi, h, wn0a, wn0b, b_n0.reshape(1, -1),
                       W_n1, b_n1.reshape(1, -1), 1000)

    n_half = n // _NC
    dxf = jnp.concatenate([xacc[0, 0:4 * n_half], xacc[1, 0:4 * n_half]])
    xa = jnp.pad(dxf, (0, 960)).reshape(-1, 128)
    xpad = jnp.pad(x, ((0, 240), (0, 1))).reshape(-1, 128)
    xnp = _xfin_call(xpad, xa)
    x_new = xnp.reshape(-1, 4)[0:n, 0:3]
    return (h_new, x_new)


# speedup vs baseline: 4.0922x; 1.0997x over previous
import functools

import jax
import jax.numpy as jnp
from jax import lax
from jax.experimental import pallas as pl
from jax.experimental.pallas import tpu as pltpu
from jax.experimental.pallas import tpu_sc as plsc

_NC = 2
_NS = 16
_NW = _NC * _NS
_L = 16
_CH = 512
_SUB = 128


def _sc_mesh():
    return plsc.VectorSubcoreMesh(
        core_axis_name="c", subcore_axis_name="s", num_cores=_NC,
        num_subcores=_NS)


def _table_body(h_ref, w_ref, out_ref):
    out_ref[...] = jnp.dot(h_ref[...], w_ref[...],
                           preferred_element_type=jnp.float32)


def _table2_body(h_ref, wc_ref, wd_ref, p_ref, q_ref):
    h = h_ref[...]
    p_ref[...] = jnp.dot(h, wc_ref[...], preferred_element_type=jnp.float32)
    q_ref[...] = jnp.dot(h, wd_ref[...], preferred_element_type=jnp.float32)


def _tables_call(h, w_c, w_d, bn):
    n = h.shape[0]
    return pl.pallas_call(
        _table2_body,
        grid=(n // bn,),
        in_specs=[
            pl.BlockSpec((bn, h.shape[1]), lambda i: (i, 0)),
            pl.BlockSpec(w_c.shape, lambda i: (0, 0)),
            pl.BlockSpec(w_d.shape, lambda i: (0, 0)),
        ],
        out_specs=[
            pl.BlockSpec((bn, 128), lambda i: (i, 0)),
            pl.BlockSpec((bn, 128), lambda i: (i, 0)),
        ],
        out_shape=[
            jax.ShapeDtypeStruct((n, 128), jnp.float32),
            jax.ShapeDtypeStruct((n, 128), jnp.float32),
        ],
    )(h, w_c, w_d)


def _efhead_call(efp, w_til, br):
    r = efp.shape[0]
    return pl.pallas_call(
        _table_body,
        grid=(r // br,),
        in_specs=[
            pl.BlockSpec((br, 128), lambda i: (i, 0)),
            pl.BlockSpec(w_til.shape, lambda i: (0, 0)),
        ],
        out_specs=pl.BlockSpec((br, w_til.shape[1]), lambda i: (i, 0)),
        out_shape=jax.ShapeDtypeStruct((r, w_til.shape[1]), jnp.float32),
    )(efp, w_til)


def _gather_call(tp, tq, dst3, src3, xflat, e_total):
    grp = 1024
    n_grp = e_total // grp
    base_p, extra_p = divmod(n_grp, _NW)
    rrows = 8 * n_grp + 8
    pk = jax.ShapeDtypeStruct((rrows, 128), jnp.float32)

    @functools.partial(
        pl.kernel,
        out_type=(
            jax.ShapeDtypeStruct((e_total, 128), jnp.float32),
            jax.ShapeDtypeStruct((e_total, 128), jnp.float32),
            pk, pk, pk, pk,
        ),
        mesh=_sc_mesh(),
        scratch_types=[
            pltpu.VMEM((8, 1, _SUB), jnp.int32),
            pltpu.VMEM((8, 1, _SUB), jnp.int32),
            pltpu.VMEM((256, 128), jnp.float32),
            pltpu.VMEM((256, 128), jnp.float32),
            pltpu.VMEM((4 * tp.shape[0],), jnp.float32),
            pltpu.VMEM((8, 128), jnp.float32),
            pltpu.VMEM((8, 128), jnp.float32),
            pltpu.VMEM((8, 128), jnp.float32),
            pltpu.VMEM((8, 128), jnp.float32),
            pltpu.SemaphoreType.DMA,
            pltpu.SemaphoreType.DMA,
            pltpu.SemaphoreType.DMA,
            pltpu.SemaphoreType.DMA,
        ],
        compiler_params=pltpu.CompilerParams(needs_layout_passes=False),
    )
    def k(tp_hbm, tq_hbm, dst_hbm, src_hbm, x_hbm,
          fd_hbm, fs_hbm, rx_hbm, ry_hbm, rz_hbm, r_hbm,
          didx, sidx, bufa, bufb, xtab, rxb, ryb, rzb, rb,
          gs0, gs1, ws0, ws1):
        cid = lax.axis_index("c")
        sid = lax.axis_index("s")
        wid = sid * _NC + cid
        trips = jnp.where(wid < extra_p, base_p + 1, base_p)

        pltpu.sync_copy(x_hbm, xtab)
        bufs = (bufa, bufb)
        gsems = (gs0, gs1)
        wsems = (ws0, ws1)

        def fire_gather(g, s):
            q, is_q = divmod(s, 2)
            tab = tq_hbm if is_q else tp_hbm
            idx = sidx if is_q else didx
            buf = bufs[s % 2]
            sem = gsems[s % 2]
            return [pltpu.async_copy(tab.at[idx.at[2 * q + j, 0]],
                                     buf.at[pl.ds(j * _SUB, _SUB)], sem)
                    for j in range(2)]

        def fire_write(g, s):
            q, is_q = divmod(s, 2)
            out = fs_hbm if is_q else fd_hbm
            return pltpu.async_copy(
                bufs[s % 2], out.at[pl.ds(g * grp + q * 256, 256)],
                wsems[s % 2])

        def rel_compute(q):
            for v in range(16):
                row = 2 * q + v // 8
                l0 = (v % 8) * _L
                d = didx[row, 0, pl.ds(l0, _L)]
                s = sidx[row, 0, pl.ds(l0, _L)]
                relc = []
                for comp in range(3):
                    xd = plsc.load_gather(xtab, [d * 4 + comp])
                    xs = plsc.load_gather(xtab, [s * 4 + comp])
                    relc.append(xd - xs)
                r2 = relc[0] * relc[0] + relc[1] * relc[1] + relc[2] * relc[2]
                rxb[row, pl.ds(l0, _L)] = relc[0]
                ryb[row, pl.ds(l0, _L)] = relc[1]
                rzb[row, pl.ds(l0, _L)] = relc[2]
                rb[row, pl.ds(l0, _L)] = r2

        def do_group(g, n_steps):
            pltpu.sync_copy(dst_hbm.at[pl.ds(8 * g, 8)], didx)
            pltpu.sync_copy(src_hbm.at[pl.ds(8 * g, 8)], sidx)
            gcps = {0: fire_gather(g, 0)}
            wcps = {}
            for s in range(n_steps):
                if s + 1 < n_steps:
                    if s - 1 >= 0:
                        wcps.pop(s - 1).wait()
                    gcps[s + 1] = fire_gather(g, s + 1)
                for cp in gcps.pop(s):
                    cp.wait()
                wcps[s] = fire_write(g, s)
                if s % 2 == 1:
                    rel_compute(s // 2)
            for s in sorted(wcps):
                wcps[s].wait()
            pltpu.sync_copy(rxb, rx_hbm.at[pl.ds(8 * g, 8)])
            pltpu.sync_copy(ryb, ry_hbm.at[pl.ds(8 * g, 8)])
            pltpu.sync_copy(rzb, rz_hbm.at[pl.ds(8 * g, 8)])
            pltpu.sync_copy(rb, r_hbm.at[pl.ds(8 * g, 8)])

        def group(t, _):
            do_group(wid + t * _NW, 8)
            return 0

        lax.fori_loop(0, trips, group, 0, unroll=False)

        @pl.when(wid == 0)
        def _():
            pltpu.sync_copy(dst_hbm.at[pl.ds(8 * n_grp, 4)],
                            didx.at[pl.ds(0, 4)])
            pltpu.sync_copy(src_hbm.at[pl.ds(8 * n_grp, 4)],
                            sidx.at[pl.ds(0, 4)])
            gcps = {0: fire_gather(n_grp, 0)}
            wcps = {}
            for s in range(4):
                if s + 1 < 4:
                    if s - 1 >= 0:
                        wcps.pop(s - 1).wait()
                    gcps[s + 1] = fire_gather(n_grp, s + 1)
                for cp in gcps.pop(s):
                    cp.wait()
                wcps[s] = fire_write(n_grp, s)
                if s % 2 == 1:
                    rel_compute(s // 2)
            for s in sorted(wcps):
                wcps[s].wait()
            pltpu.sync_copy(rxb, rx_hbm.at[pl.ds(8 * n_grp, 8)])
            pltpu.sync_copy(ryb, ry_hbm.at[pl.ds(8 * n_grp, 8)])
            pltpu.sync_copy(rzb, rz_hbm.at[pl.ds(8 * n_grp, 8)])
            pltpu.sync_copy(rb, r_hbm.at[pl.ds(8 * n_grp, 8)])

    return k(tp, tq, dst3, src3, xflat)


def _edge_body(ng, r_coeff, r_step, gpb, fd_ref, fs_ref, g0_ref, rp_ref,
               wb_ref, be0_ref, we1_ref, be1_ref, winf_ref, binf_ref,
               wx0_ref, bx0_ref, wx1_ref, bx1_ref, out_ref, xw_ref):
    bc = gpb * 128
    rpk = rp_ref[0]
    rt = jnp.transpose(rpk)
    r = jnp.concatenate([rt[:, g:g + 1] for g in range(gpb)], axis=0)
    offs = lax.broadcasted_iota(jnp.int32, (1, ng), 1).astype(jnp.float32)
    offs = offs * r_step
    r_feat = jnp.exp(r_coeff * (r - offs) ** 2)
    pre = (fd_ref[...] + fs_ref[...] + g0_ref[...] + be0_ref[...]
           + jnp.dot(r_feat, wb_ref[...], preferred_element_type=jnp.float32))
    u = jnp.maximum(pre, 0.0)
    mij = jnp.maximum(
        jnp.dot(u, we1_ref[...], preferred_element_type=jnp.float32)
        + be1_ref[...], 0.0)
    z = jnp.sum(mij * winf_ref[...], axis=-1, keepdims=True) + binf_ref[...]
    eij = 1.0 / (1.0 + jnp.exp(-z))
    t = jnp.maximum(
        jnp.dot(mij, wx0_ref[...], preferred_element_type=jnp.float32)
        + bx0_ref[...], 0.0)
    xw = jnp.sum(t * wx1_ref[...], axis=-1, keepdims=True) + bx1_ref[...]
    out_ref[...] = mij * eij
    xw_ref[...] = jnp.reshape(xw, (1, gpb, 128))


def _edge_call(fd, fs, g0, rp3, wb, b_e0, w_e1, b_e1, w_inf, b_inf,
               w_x0, b_x0, w_x1, b_x1, ng, r_coeff, r_step, bc):
    e_total = fd.shape[0]
    gpb = bc // 128
    nb = e_total // bc
    full = lambda a: pl.BlockSpec(a.shape, lambda i: tuple(0 for _ in a.shape))
    return pl.pallas_call(
        functools.partial(_edge_body, ng, r_coeff, r_step, gpb),
        grid=(nb,),
        in_specs=[
            pl.BlockSpec((bc, 128), lambda i: (i, 0)),
            pl.BlockSpec((bc, 128), lambda i: (i, 0)),
            pl.BlockSpec((bc, 128), lambda i: (i, 0)),
            pl.BlockSpec((1, gpb, 128), lambda i: (i, 0, 0)),
            full(wb), full(b_e0), full(w_e1), full(b_e1), full(w_inf),
            full(b_inf), full(w_x0), full(b_x0), full(w_x1), full(b_x1),
        ],
        out_specs=[
            pl.BlockSpec((bc, 128), lambda i: (i, 0)),
            pl.BlockSpec((1, gpb, 128), lambda i: (i, 0, 0)),
        ],
        out_shape=[
            jax.ShapeDtypeStruct((e_total, 128), jnp.float32),
            jax.ShapeDtypeStruct((nb, gpb, 128), jnp.float32),
        ],
    )(fd, fs, g0, rp3, wb, b_e0, w_e1, b_e1, w_inf, b_inf, w_x0, b_x0,
      w_x1, b_x1)


def _scatter_call(gated, dst3, xwp, rxp, ryp, rzp, zeros_nf, n_total,
                  e_total):
    n_grp = e_total // 1024
    base_p, extra_p = divmod(n_grp, _NS)
    n_half = n_total // _NC
    rpt = 312
    xacc_len = 1280 * _NS

    @functools.partial(
        pl.kernel,
        out_type=(
            jax.ShapeDtypeStruct((_NC, n_half, 128), jnp.float32),
            jax.ShapeDtypeStruct((_NC, xacc_len), jnp.float32),
        ),
        mesh=_sc_mesh(),
        scratch_types=[
            pltpu.VMEM((8, 1, _SUB), jnp.int32),
            pltpu.VMEM((8, 1, _SUB), jnp.int32),
            pltpu.VMEM((256, 128), jnp.float32),
            pltpu.VMEM((256, 128), jnp.float32),
            pltpu.VMEM((8, 128), jnp.float32),
            pltpu.VMEM((8, 128), jnp.float32),
            pltpu.VMEM((8, 128), jnp.float32),
            pltpu.VMEM((8, 128), jnp.float32),
            pltpu.VMEM((4096,), jnp.float32),
            pltpu.VMEM((32, 1, _SUB), jnp.int32),
            pltpu.VMEM((1280,), jnp.float32),
            pltpu.VMEM_SHARED((n_half, 128), jnp.float32),
            pltpu.VMEM_SHARED((xacc_len,), jnp.float32),
            pltpu.SemaphoreType.DMA,
            pltpu.SemaphoreType.DMA,
            pltpu.SemaphoreType.DMA,
            pltpu.SemaphoreType.DMA,
            pltpu.SemaphoreType.DMA,
        ],
        compiler_params=pltpu.CompilerParams(needs_layout_passes=False),
    )
    def k(g_hbm, dst_hbm, xw_hbm, rx_hbm, ry_hbm, rz_hbm, z_hbm,
          acc_hbm, xacc_hbm,
          didx, fidx, growa, growb, xwb, rxb, ryb, rzb, vals, idxs, zbuf,
          acc, xacc, ls0, ls1, as0, as1, esem):
        asems = (as0, as1)
        cid = lax.axis_index("c")
        sid = lax.axis_index("s")
        nbase = cid * n_half
        trips = jnp.where(sid < extra_p, base_p + 1, base_p)

        def zloop(i, _):
            zbuf[pl.ds(i * _L, _L)] = jnp.zeros((_L,), jnp.float32)
            return 0
        lax.fori_loop(0, 1280 // _L, zloop, 0, unroll=False)
        pltpu.sync_copy(zbuf, xacc.at[pl.ds(sid * 1280, 1280)])

        @pl.when(sid < _NS - 1)
        def _():
            pltpu.sync_copy(z_hbm.at[pl.ds(0, rpt)],
                            acc.at[pl.ds(sid * rpt, rpt)])

        @pl.when(sid == _NS - 1)
        def _():
            pltpu.sync_copy(z_hbm.at[pl.ds(0, 320)],
                            acc.at[pl.ds((_NS - 1) * rpt, 320)])

        plsc.subcore_barrier()

        def build_filtered(n_rows):
            for v in range(8 * n_rows):
                row = v // 8
                l0 = (v % 8) * _L
                d = didx[row, 0, pl.ds(l0, _L)] - nbase
                valid = (d >= 0) & (d < n_half)
                fidx[row, 0, pl.ds(l0, _L)] = jnp.where(valid, d, -1)
                xw = xwb[row, pl.ds(l0, _L)]
                base = v * 64
                pos0 = lax.iota(jnp.int32, _L) * 4
                for comp, rbuf in ((0, rxb), (1, ryb), (2, rzb)):
                    val = rbuf[row, pl.ds(l0, _L)] * xw
                    pos = pos0 + (base + comp)
                    plsc.store_scatter(vals, [pos], val)
                    plsc.store_scatter(
                        idxs, [pos // _SUB,
                               jnp.zeros((_L,), jnp.int32),
                               lax.rem(pos, _SUB)],
                        jnp.where(valid, d * 4 + comp, -1))
                pos = pos0 + (base + 3)
                plsc.store_scatter(
                    idxs, [pos // _SUB,
                           jnp.zeros((_L,), jnp.int32),
                           lax.rem(pos, _SUB)],
                    jnp.full((_L,), -1, jnp.int32))

        def do_group(g, n_sub):
            pltpu.sync_copy(dst_hbm.at[pl.ds(8 * g, 2 * n_sub)],
                            didx.at[pl.ds(0, 2 * n_sub)])
            pltpu.sync_copy(xw_hbm.at[pl.ds(8 * g, 8)], xwb)
            pltpu.sync_copy(rx_hbm.at[pl.ds(8 * g, 8)], rxb)
            pltpu.sync_copy(ry_hbm.at[pl.ds(8 * g, 8)], ryb)
            pltpu.sync_copy(rz_hbm.at[pl.ds(8 * g, 8)], rzb)
            bufs = (growa, growb)
            lsems = (ls0, ls1)

            def fire_load(s):
                return pltpu.async_copy(
                    g_hbm.at[pl.ds(g * 1024 + s * 256, 256)],
                    bufs[s % 2], lsems[s % 2])

            build_filtered(2 * n_sub)
            ecps = [pltpu.async_copy(
                vals.at[pl.ds(t * _SUB, _SUB)],
                xacc.at[plsc.Indices(idxs.at[t, 0], ignored_value=-1)],
                esem, add=True) for t in range(8 * n_sub)]
            lcps = {0: fire_load(0)}
            acps = {}
            for s in range(n_sub):
                if s + 1 < n_sub:
                    if s - 1 >= 0:
                        for cp in acps.pop(s - 1):
                            cp.wait()
                    lcps[s + 1] = fire_load(s + 1)
                lcps.pop(s).wait()
                acps[s] = [pltpu.async_copy(
                    bufs[s % 2].at[pl.ds(j * _SUB, _SUB)],
                    acc.at[plsc.Indices(fidx.at[2 * s + j, 0],
                                        ignored_value=-1)],
                    asems[s % 2], add=True) for j in range(2)]
            for s in sorted(acps):
                for cp in acps[s]:
                    cp.wait()
            for cp in ecps:
                cp.wait()

        def group(t, _):
            do_group(sid + t * _NS, 4)
            return 0

        lax.fori_loop(0, trips, group, 0, unroll=False)

        @pl.when(sid == 0)
        def _():
            do_group(n_grp, 2)

        plsc.subcore_barrier()

        @pl.when(sid < _NS - 1)
        def _():
            pltpu.sync_copy(acc.at[pl.ds(sid * rpt, rpt)],
                            acc_hbm.at[cid, pl.ds(sid * rpt, rpt)])

        @pl.when(sid == _NS - 1)
        def _():
            pltpu.sync_copy(acc.at[pl.ds((_NS - 1) * rpt, 320)],
                            acc_hbm.at[cid, pl.ds((_NS - 1) * rpt, 320)])

        pltpu.sync_copy(xacc.at[pl.ds(sid * 1280, 1280)],
                        xacc_hbm.at[cid, pl.ds(sid * 1280, 1280)])

    return k(gated, dst3, xwp, rxp, ryp, rzp, zeros_nf)


def _node_body(mi_ref, h_ref, wn0a_ref, wn0b_ref, bn0_ref,
               wn1_ref, bn1_ref, hout_ref):
    mi = mi_ref[0]
    h = h_ref[...]
    u = jnp.maximum(
        jnp.dot(mi, wn0a_ref[...], preferred_element_type=jnp.float32)
        + jnp.dot(h, wn0b_ref[...], preferred_element_type=jnp.float32)
        + bn0_ref[...], 0.0)
    hout_ref[...] = h + jnp.dot(
        u, wn1_ref[...], preferred_element_type=jnp.float32) + bn1_ref[...]


def _node_call(acc, h, wn0a, wn0b, b_n0, w_n1, b_n1, bn):
    n = h.shape[0]
    hpb = (n // _NC) // bn
    full = lambda a: pl.BlockSpec(a.shape, lambda i: tuple(0 for _ in a.shape))
    return pl.pallas_call(
        _node_body,
        grid=(n // bn,),
        in_specs=[
            pl.BlockSpec((1, bn, 128), lambda i: (i // hpb, i % hpb, 0)),
            pl.BlockSpec((bn, 128), lambda i: (i, 0)),
            full(wn0a), full(wn0b), full(b_n0), full(w_n1), full(b_n1),
        ],
        out_specs=pl.BlockSpec((bn, 128), lambda i: (i, 0)),
        out_shape=jax.ShapeDtypeStruct((n, 128), jnp.float32),
    )(acc, h, wn0a, wn0b, b_n0, w_n1, b_n1)


def _xfin_body(xp_ref, a0_ref, out_ref):
    out_ref[...] = xp_ref[...] + a0_ref[...]


def _xfin_call(xp, a0):
    spec = pl.BlockSpec(xp.shape, lambda: (0, 0))
    return pl.pallas_call(
        _xfin_body,
        in_specs=[spec, spec],
        out_specs=spec,
        out_shape=jax.ShapeDtypeStruct(xp.shape, jnp.float32),
    )(xp, a0)


def kernel(h, x, edge_feat, edge_index, W_e0, b_e0, W_e1, b_e1, W_inf, b_inf,
           W_x0, b_x0, W_x1, b_x1, W_n0, b_n0, W_n1, b_n1):
    n, hd = h.shape
    e = edge_index.shape[1]
    ef = edge_feat.shape[1]
    ng = W_e0.shape[0] - 2 * hd - ef
    r_step = 100.0 / (ng - 1)
    r_coeff = -0.5 / r_step ** 2
    bc = 2560
    gpb = bc // 128

    src = edge_index[0]
    dst = edge_index[1]
    dst3 = dst.reshape(e // _SUB, 1, _SUB)
    src3 = src.reshape(e // _SUB, 1, _SUB)

    w_a = W_e0[0:ef]
    w_b = W_e0[ef:ef + ng]
    w_c = W_e0[ef + ng:ef + ng + hd]
    w_d = W_e0[ef + ng + hd:]
    w_til = jnp.kron(jnp.eye(8, dtype=jnp.float32), w_a)
    wn0a = W_n0[0:hd]
    wn0b = W_n0[hd:]

    tp, tq = _tables_call(h, w_c, w_d, 1000)
    xflat = jnp.pad(x, ((0, 0), (0, 1))).reshape(-1)

    efp = edge_feat.reshape(e // 8, 128)
    g0 = _efhead_call(efp, w_til, 1000).reshape(e, 128)

    fd, fs, rxp, ryp, rzp, rp = _gather_call(tp, tq, dst3, src3, xflat, e)
    rp3 = rp[0:e // 128].reshape(e // bc, gpb, 128)

    gated, xw3 = _edge_call(
        fd, fs, g0, rp3, w_b, b_e0.reshape(1, -1), W_e1, b_e1.reshape(1, -1),
        W_inf.reshape(1, -1), b_inf.reshape(1, 1), W_x0, b_x0.reshape(1, -1),
        W_x1.reshape(1, -1), b_x1.reshape(1, 1), ng, r_coeff, r_step, bc)

    xwp = jnp.pad(xw3.reshape(e // 128, 128), ((0, rxp.shape[0] - e // 128),
                                               (0, 0)))

    zeros_nf = jnp.zeros((n, 128), jnp.float32)
    acc, xacc = _scatter_call(gated, dst3, xwp, rxp, ryp, rzp, zeros_nf, n, e)

    h_new = _node_call(acc, h, wn0a, wn0b, b_n0.reshape(1, -1),
                       W_n1, b_n1.reshape(1, -1), 1000)

    n_half = n // _NC
    dxf = jnp.concatenate([xacc[0, 0:4 * n_half], xacc[1, 0:4 * n_half]])
    xa = jnp.pad(dxf, (0, 960)).reshape(-1, 128)
    xpad = jnp.pad(x, ((0, 240), (0, 1))).reshape(-1, 128)
    xnp = _xfin_call(xpad, xa)
    x_new = xnp.reshape(-1, 4)[0:n, 0:3]
    return (h_new, x_new)


# speedup vs baseline: 4.5676x; 1.1162x over previous
import functools

import jax
import jax.numpy as jnp
from jax import lax
from jax.experimental import pallas as pl
from jax.experimental.pallas import tpu as pltpu
from jax.experimental.pallas import tpu_sc as plsc

_NC = 2
_NS = 16
_NW = _NC * _NS
_L = 16
_CH = 512
_SUB = 128


def _sc_mesh():
    return plsc.VectorSubcoreMesh(
        core_axis_name="c", subcore_axis_name="s", num_cores=_NC,
        num_subcores=_NS)


def _table_body(h_ref, w_ref, out_ref):
    out_ref[...] = jnp.dot(h_ref[...], w_ref[...],
                           preferred_element_type=jnp.float32)


def _table2_body(h_ref, wc_ref, wd_ref, p_ref, q_ref):
    h = h_ref[...]
    p_ref[...] = jnp.dot(h, wc_ref[...], preferred_element_type=jnp.float32)
    q_ref[...] = jnp.dot(h, wd_ref[...], preferred_element_type=jnp.float32)


def _tables_call(h, w_c, w_d, bn):
    n = h.shape[0]
    return pl.pallas_call(
        _table2_body,
        grid=(n // bn,),
        in_specs=[
            pl.BlockSpec((bn, h.shape[1]), lambda i: (i, 0)),
            pl.BlockSpec(w_c.shape, lambda i: (0, 0)),
            pl.BlockSpec(w_d.shape, lambda i: (0, 0)),
        ],
        out_specs=[
            pl.BlockSpec((bn, 128), lambda i: (i, 0)),
            pl.BlockSpec((bn, 128), lambda i: (i, 0)),
        ],
        out_shape=[
            jax.ShapeDtypeStruct((n, 128), jnp.float32),
            jax.ShapeDtypeStruct((n, 128), jnp.float32),
        ],
    )(h, w_c, w_d)


def _efhead_call(efp, w_til, br):
    r = efp.shape[0]
    return pl.pallas_call(
        _table_body,
        grid=(r // br,),
        in_specs=[
            pl.BlockSpec((br, 128), lambda i: (i, 0)),
            pl.BlockSpec(w_til.shape, lambda i: (0, 0)),
        ],
        out_specs=pl.BlockSpec((br, w_til.shape[1]), lambda i: (i, 0)),
        out_shape=jax.ShapeDtypeStruct((r, w_til.shape[1]), jnp.float32),
    )(efp, w_til)


def _gather_call(tp, tq, dst3, src3, xflat, e_total):
    grp = 1024
    n_grp = e_total // grp
    base_p, extra_p = divmod(n_grp, _NW)
    rrows = 8 * n_grp + 8
    pk = jax.ShapeDtypeStruct((rrows, 128), jnp.float32)

    @functools.partial(
        pl.kernel,
        out_type=(
            jax.ShapeDtypeStruct((e_total, 128), jnp.float32),
            jax.ShapeDtypeStruct((e_total, 128), jnp.float32),
            pk, pk, pk, pk,
        ),
        mesh=_sc_mesh(),
        scratch_types=[
            pltpu.VMEM((8, 1, _SUB), jnp.int32),
            pltpu.VMEM((8, 1, _SUB), jnp.int32),
            pltpu.VMEM((256, 128), jnp.float32),
            pltpu.VMEM((256, 128), jnp.float32),
            pltpu.VMEM((4 * tp.shape[0],), jnp.float32),
            pltpu.VMEM((8, 128), jnp.float32),
            pltpu.VMEM((8, 128), jnp.float32),
            pltpu.VMEM((8, 128), jnp.float32),
            pltpu.VMEM((8, 128), jnp.float32),
            pltpu.SemaphoreType.DMA,
            pltpu.SemaphoreType.DMA,
            pltpu.SemaphoreType.DMA,
            pltpu.SemaphoreType.DMA,
            pltpu.SemaphoreType.DMA,
            pltpu.SemaphoreType.DMA,
        ],
        compiler_params=pltpu.CompilerParams(needs_layout_passes=False),
    )
    def k(tp_hbm, tq_hbm, dst_hbm, src_hbm, x_hbm,
          fd_hbm, fs_hbm, rx_hbm, ry_hbm, rz_hbm, r_hbm,
          didx, sidx, bufa, bufb, xtab, rxb, ryb, rzb, rb,
          gs0, gs1, ws0, ws1, isem, rwsem):
        cid = lax.axis_index("c")
        sid = lax.axis_index("s")
        wid = sid * _NC + cid
        trips = jnp.where(wid < extra_p, base_p + 1, base_p)

        pltpu.sync_copy(x_hbm, xtab)
        bufs = (bufa, bufb)
        gsems = (gs0, gs1)
        wsems = (ws0, ws1)

        def fire_gather(g, s):
            q, is_q = divmod(s, 2)
            tab = tq_hbm if is_q else tp_hbm
            idx = sidx if is_q else didx
            buf = bufs[s % 2]
            sem = gsems[s % 2]
            return [pltpu.async_copy(tab.at[idx.at[2 * q + j, 0]],
                                     buf.at[pl.ds(j * _SUB, _SUB)], sem)
                    for j in range(2)]

        def fire_write(g, s):
            q, is_q = divmod(s, 2)
            out = fs_hbm if is_q else fd_hbm
            return pltpu.async_copy(
                bufs[s % 2], out.at[pl.ds(g * grp + q * 256, 256)],
                wsems[s % 2])

        def rel_compute(q):
            for v in range(16):
                row = 2 * q + v // 8
                l0 = (v % 8) * _L
                d = didx[row, 0, pl.ds(l0, _L)]
                s = sidx[row, 0, pl.ds(l0, _L)]
                relc = []
                for comp in range(3):
                    xd = plsc.load_gather(xtab, [d * 4 + comp])
                    xs = plsc.load_gather(xtab, [s * 4 + comp])
                    relc.append(xd - xs)
                r2 = relc[0] * relc[0] + relc[1] * relc[1] + relc[2] * relc[2]
                rxb[row, pl.ds(l0, _L)] = relc[0]
                ryb[row, pl.ds(l0, _L)] = relc[1]
                rzb[row, pl.ds(l0, _L)] = relc[2]
                rb[row, pl.ds(l0, _L)] = r2

        def do_group(g, n_steps, pipelined):
            icps = [pltpu.async_copy(dst_hbm.at[pl.ds(8 * g, 8)], didx,
                                     isem),
                    pltpu.async_copy(src_hbm.at[pl.ds(8 * g, 8)], sidx,
                                     isem)]
            for cp in icps:
                cp.wait()
            gcps = {0: fire_gather(g, 0)}
            wcps = {}
            for s in range(n_steps):
                if s + 1 < n_steps:
                    if s - 1 >= 0:
                        wcps.pop(s - 1).wait()
                    gcps[s + 1] = fire_gather(g, s + 1)
                for cp in gcps.pop(s):
                    cp.wait()
                wcps[s] = fire_write(g, s)
                if s % 2 == 1:
                    rel_compute(s // 2)
            for s in sorted(wcps):
                wcps[s].wait()
            if pipelined:
                pltpu.async_copy(rxb, rx_hbm.at[pl.ds(8 * g, 8)], rwsem)
                pltpu.async_copy(ryb, ry_hbm.at[pl.ds(8 * g, 8)], rwsem)
                pltpu.async_copy(rzb, rz_hbm.at[pl.ds(8 * g, 8)], rwsem)
                pltpu.async_copy(rb, r_hbm.at[pl.ds(8 * g, 8)], rwsem)
            else:
                pltpu.sync_copy(rxb, rx_hbm.at[pl.ds(8 * g, 8)])
                pltpu.sync_copy(ryb, ry_hbm.at[pl.ds(8 * g, 8)])
                pltpu.sync_copy(rzb, rz_hbm.at[pl.ds(8 * g, 8)])
                pltpu.sync_copy(rb, r_hbm.at[pl.ds(8 * g, 8)])

        def drain_rel():
            for _ in range(4):
                pltpu.make_async_copy(rxb, rx_hbm.at[pl.ds(0, 8)],
                                      rwsem).wait()

        def group(t, _):
            @pl.when(t > 0)
            def _():
                drain_rel()
            do_group(wid + t * _NW, 8, True)
            return 0

        lax.fori_loop(0, trips, group, 0, unroll=False)
        drain_rel()

        @pl.when(wid == 0)
        def _():
            pltpu.sync_copy(dst_hbm.at[pl.ds(8 * n_grp, 4)],
                            didx.at[pl.ds(0, 4)])
            pltpu.sync_copy(src_hbm.at[pl.ds(8 * n_grp, 4)],
                            sidx.at[pl.ds(0, 4)])
            gcps = {0: fire_gather(n_grp, 0)}
            wcps = {}
            for s in range(4):
                if s + 1 < 4:
                    if s - 1 >= 0:
                        wcps.pop(s - 1).wait()
                    gcps[s + 1] = fire_gather(n_grp, s + 1)
                for cp in gcps.pop(s):
                    cp.wait()
                wcps[s] = fire_write(n_grp, s)
                if s % 2 == 1:
                    rel_compute(s // 2)
            for s in sorted(wcps):
                wcps[s].wait()
            pltpu.sync_copy(rxb, rx_hbm.at[pl.ds(8 * n_grp, 8)])
            pltpu.sync_copy(ryb, ry_hbm.at[pl.ds(8 * n_grp, 8)])
            pltpu.sync_copy(rzb, rz_hbm.at[pl.ds(8 * n_grp, 8)])
            pltpu.sync_copy(rb, r_hbm.at[pl.ds(8 * n_grp, 8)])

    return k(tp, tq, dst3, src3, xflat)


def _edge_body(ng, r_coeff, r_step, gpb, fd_ref, fs_ref, g0_ref, rp_ref,
               wb_ref, be0_ref, we1_ref, be1_ref, winf_ref, binf_ref,
               wx0_ref, bx0_ref, wx1_ref, bx1_ref, out_ref, xw_ref):
    bc = gpb * 128
    rpk = rp_ref[0]
    rt = jnp.transpose(rpk)
    r = jnp.concatenate([rt[:, g:g + 1] for g in range(gpb)], axis=0)
    offs = lax.broadcasted_iota(jnp.int32, (1, ng), 1).astype(jnp.float32)
    offs = offs * r_step
    r_feat = jnp.exp(r_coeff * (r - offs) ** 2)
    pre = (fd_ref[...] + fs_ref[...] + g0_ref[...] + be0_ref[...]
           + jnp.dot(r_feat, wb_ref[...], preferred_element_type=jnp.float32))
    u = jnp.maximum(pre, 0.0)
    mij = jnp.maximum(
        jnp.dot(u, we1_ref[...], preferred_element_type=jnp.float32)
        + be1_ref[...], 0.0)
    z = jnp.dot(mij, winf_ref[...],
                preferred_element_type=jnp.float32) + binf_ref[...]
    eij = 1.0 / (1.0 + jnp.exp(-z))
    t = jnp.maximum(
        jnp.dot(mij, wx0_ref[...], preferred_element_type=jnp.float32)
        + bx0_ref[...], 0.0)
    xw = jnp.dot(t, wx1_ref[...],
                 preferred_element_type=jnp.float32) + bx1_ref[...]
    out_ref[...] = mij * eij
    xw_ref[...] = jnp.reshape(xw, (1, gpb, 128))


def _edge_call(fd, fs, g0, rp3, wb, b_e0, w_e1, b_e1, w_inf, b_inf,
               w_x0, b_x0, w_x1, b_x1, ng, r_coeff, r_step, bc):
    e_total = fd.shape[0]
    gpb = bc // 128
    nb = e_total // bc
    full = lambda a: pl.BlockSpec(a.shape, lambda i: tuple(0 for _ in a.shape))
    return pl.pallas_call(
        functools.partial(_edge_body, ng, r_coeff, r_step, gpb),
        grid=(nb,),
        in_specs=[
            pl.BlockSpec((bc, 128), lambda i: (i, 0)),
            pl.BlockSpec((bc, 128), lambda i: (i, 0)),
            pl.BlockSpec((bc, 128), lambda i: (i, 0)),
            pl.BlockSpec((1, gpb, 128), lambda i: (i, 0, 0)),
            full(wb), full(b_e0), full(w_e1), full(b_e1), full(w_inf),
            full(b_inf), full(w_x0), full(b_x0), full(w_x1), full(b_x1),
        ],
        out_specs=[
            pl.BlockSpec((bc, 128), lambda i: (i, 0)),
            pl.BlockSpec((1, gpb, 128), lambda i: (i, 0, 0)),
        ],
        out_shape=[
            jax.ShapeDtypeStruct((e_total, 128), jnp.float32),
            jax.ShapeDtypeStruct((nb, gpb, 128), jnp.float32),
        ],
    )(fd, fs, g0, rp3, wb, b_e0, w_e1, b_e1, w_inf, b_inf, w_x0, b_x0,
      w_x1, b_x1)


def _scatter_call(gated, dst3, xwp, rxp, ryp, rzp, zeros_nf, n_total,
                  e_total):
    n_grp = e_total // 1024
    base_p, extra_p = divmod(n_grp, _NS)
    n_half = n_total // _NC
    rpt = 312
    xacc_len = 1280 * _NS

    @functools.partial(
        pl.kernel,
        out_type=(
            jax.ShapeDtypeStruct((_NC, n_half, 128), jnp.float32),
            jax.ShapeDtypeStruct((_NC, xacc_len), jnp.float32),
        ),
        mesh=_sc_mesh(),
        scratch_types=[
            pltpu.VMEM((8, 1, _SUB), jnp.int32),
            pltpu.VMEM((8, 1, _SUB), jnp.int32),
            pltpu.VMEM((256, 128), jnp.float32),
            pltpu.VMEM((256, 128), jnp.float32),
            pltpu.VMEM((8, 128), jnp.float32),
            pltpu.VMEM((8, 128), jnp.float32),
            pltpu.VMEM((8, 128), jnp.float32),
            pltpu.VMEM((8, 128), jnp.float32),
            pltpu.VMEM((4096,), jnp.float32),
            pltpu.VMEM((32, 1, _SUB), jnp.int32),
            pltpu.VMEM((1280,), jnp.float32),
            pltpu.VMEM_SHARED((n_half, 128), jnp.float32),
            pltpu.VMEM_SHARED((xacc_len,), jnp.float32),
            pltpu.SemaphoreType.DMA,
            pltpu.SemaphoreType.DMA,
            pltpu.SemaphoreType.DMA,
            pltpu.SemaphoreType.DMA,
            pltpu.SemaphoreType.DMA,
        ],
        compiler_params=pltpu.CompilerParams(needs_layout_passes=False),
    )
    def k(g_hbm, dst_hbm, xw_hbm, rx_hbm, ry_hbm, rz_hbm, z_hbm,
          acc_hbm, xacc_hbm,
          didx, fidx, growa, growb, xwb, rxb, ryb, rzb, vals, idxs, zbuf,
          acc, xacc, ls0, ls1, as0, as1, esem):
        asems = (as0, as1)
        cid = lax.axis_index("c")
        sid = lax.axis_index("s")
        nbase = cid * n_half
        trips = jnp.where(sid < extra_p, base_p + 1, base_p)

        def zloop(i, _):
            zbuf[pl.ds(i * _L, _L)] = jnp.zeros((_L,), jnp.float32)
            return 0
        lax.fori_loop(0, 1280 // _L, zloop, 0, unroll=False)
        pltpu.sync_copy(zbuf, xacc.at[pl.ds(sid * 1280, 1280)])

        @pl.when(sid < _NS - 1)
        def _():
            pltpu.sync_copy(z_hbm.at[pl.ds(0, rpt)],
                            acc.at[pl.ds(sid * rpt, rpt)])

        @pl.when(sid == _NS - 1)
        def _():
            pltpu.sync_copy(z_hbm.at[pl.ds(0, 320)],
                            acc.at[pl.ds((_NS - 1) * rpt, 320)])

        plsc.subcore_barrier()

        def build_filtered(n_rows):
            for v in range(8 * n_rows):
                row = v // 8
                l0 = (v % 8) * _L
                d = didx[row, 0, pl.ds(l0, _L)] - nbase
                valid = (d >= 0) & (d < n_half)
                fidx[row, 0, pl.ds(l0, _L)] = jnp.where(valid, d, -1)
                xw = xwb[row, pl.ds(l0, _L)]
                base = v * 64
                pos0 = lax.iota(jnp.int32, _L) * 4
                for comp, rbuf in ((0, rxb), (1, ryb), (2, rzb)):
                    val = rbuf[row, pl.ds(l0, _L)] * xw
                    pos = pos0 + (base + comp)
                    plsc.store_scatter(vals, [pos], val)
                    plsc.store_scatter(
                        idxs, [pos // _SUB,
                               jnp.zeros((_L,), jnp.int32),
                               lax.rem(pos, _SUB)],
                        jnp.where(valid, d * 4 + comp, -1))
                pos = pos0 + (base + 3)
                plsc.store_scatter(
                    idxs, [pos // _SUB,
                           jnp.zeros((_L,), jnp.int32),
                           lax.rem(pos, _SUB)],
                    jnp.full((_L,), -1, jnp.int32))

        def do_group(g, n_sub):
            bufs = (growa, growb)
            lsems = (ls0, ls1)

            def fire_load(s):
                return pltpu.async_copy(
                    g_hbm.at[pl.ds(g * 1024 + s * 256, 256)],
                    bufs[s % 2], lsems[s % 2])

            hcps = [
                pltpu.async_copy(dst_hbm.at[pl.ds(8 * g, 2 * n_sub)],
                                 didx.at[pl.ds(0, 2 * n_sub)], esem),
                pltpu.async_copy(xw_hbm.at[pl.ds(8 * g, 8)], xwb, esem),
                pltpu.async_copy(rx_hbm.at[pl.ds(8 * g, 8)], rxb, esem),
                pltpu.async_copy(ry_hbm.at[pl.ds(8 * g, 8)], ryb, esem),
                pltpu.async_copy(rz_hbm.at[pl.ds(8 * g, 8)], rzb, esem),
            ]
            lcps = {0: fire_load(0)}
            for cp in hcps:
                cp.wait()
            build_filtered(2 * n_sub)
            ecps = [pltpu.async_copy(
                vals.at[pl.ds(t * _SUB, _SUB)],
                xacc.at[plsc.Indices(idxs.at[t, 0], ignored_value=-1)],
                esem, add=True) for t in range(8 * n_sub)]
            acps = {}
            for s in range(n_sub):
                if s + 1 < n_sub:
                    if s - 1 >= 0:
                        for cp in acps.pop(s - 1):
                            cp.wait()
                    lcps[s + 1] = fire_load(s + 1)
                lcps.pop(s).wait()
                acps[s] = [pltpu.async_copy(
                    bufs[s % 2].at[pl.ds(j * _SUB, _SUB)],
                    acc.at[plsc.Indices(fidx.at[2 * s + j, 0],
                                        ignored_value=-1)],
                    asems[s % 2], add=True) for j in range(2)]
            for s in sorted(acps):
                for cp in acps[s]:
                    cp.wait()
            for cp in ecps:
                cp.wait()

        def group(t, _):
            do_group(sid + t * _NS, 4)
            return 0

        lax.fori_loop(0, trips, group, 0, unroll=False)

        @pl.when(sid == 0)
        def _():
            do_group(n_grp, 2)

        plsc.subcore_barrier()

        @pl.when(sid < _NS - 1)
        def _():
            pltpu.sync_copy(acc.at[pl.ds(sid * rpt, rpt)],
                            acc_hbm.at[cid, pl.ds(sid * rpt, rpt)])

        @pl.when(sid == _NS - 1)
        def _():
            pltpu.sync_copy(acc.at[pl.ds((_NS - 1) * rpt, 320)],
                            acc_hbm.at[cid, pl.ds((_NS - 1) * rpt, 320)])

        pltpu.sync_copy(xacc.at[pl.ds(sid * 1280, 1280)],
                        xacc_hbm.at[cid, pl.ds(sid * 1280, 1280)])

    return k(gated, dst3, xwp, rxp, ryp, rzp, zeros_nf)


def _node_body(mi_ref, h_ref, wn0a_ref, wn0b_ref, bn0_ref,
               wn1_ref, bn1_ref, hout_ref):
    mi = mi_ref[0]
    h = h_ref[...]
    u = jnp.maximum(
        jnp.dot(mi, wn0a_ref[...], preferred_element_type=jnp.float32)
        + jnp.dot(h, wn0b_ref[...], preferred_element_type=jnp.float32)
        + bn0_ref[...], 0.0)
    hout_ref[...] = h + jnp.dot(
        u, wn1_ref[...], preferred_element_type=jnp.float32) + bn1_ref[...]


def _node_call(acc, h, wn0a, wn0b, b_n0, w_n1, b_n1, bn):
    n = h.shape[0]
    hpb = (n // _NC) // bn
    full = lambda a: pl.BlockSpec(a.shape, lambda i: tuple(0 for _ in a.shape))
    return pl.pallas_call(
        _node_body,
        grid=(n // bn,),
        in_specs=[
            pl.BlockSpec((1, bn, 128), lambda i: (i // hpb, i % hpb, 0)),
            pl.BlockSpec((bn, 128), lambda i: (i, 0)),
            full(wn0a), full(wn0b), full(b_n0), full(w_n1), full(b_n1),
        ],
        out_specs=pl.BlockSpec((bn, 128), lambda i: (i, 0)),
        out_shape=jax.ShapeDtypeStruct((n, 128), jnp.float32),
    )(acc, h, wn0a, wn0b, b_n0, w_n1, b_n1)


def _xfin_body(xp_ref, a0_ref, out_ref):
    out_ref[...] = xp_ref[...] + a0_ref[...]


def _xfin_call(xp, a0):
    spec = pl.BlockSpec(xp.shape, lambda: (0, 0))
    return pl.pallas_call(
        _xfin_body,
        in_specs=[spec, spec],
        out_specs=spec,
        out_shape=jax.ShapeDtypeStruct(xp.shape, jnp.float32),
    )(xp, a0)


def kernel(h, x, edge_feat, edge_index, W_e0, b_e0, W_e1, b_e1, W_inf, b_inf,
           W_x0, b_x0, W_x1, b_x1, W_n0, b_n0, W_n1, b_n1):
    n, hd = h.shape
    e = edge_index.shape[1]
    ef = edge_feat.shape[1]
    ng = W_e0.shape[0] - 2 * hd - ef
    r_step = 100.0 / (ng - 1)
    r_coeff = -0.5 / r_step ** 2
    bc = 2560
    gpb = bc // 128

    src = edge_index[0]
    dst = edge_index[1]
    dst3 = dst.reshape(e // _SUB, 1, _SUB)
    src3 = src.reshape(e // _SUB, 1, _SUB)

    w_a = W_e0[0:ef]
    w_b = W_e0[ef:ef + ng]
    w_c = W_e0[ef + ng:ef + ng + hd]
    w_d = W_e0[ef + ng + hd:]
    w_til = jnp.kron(jnp.eye(8, dtype=jnp.float32), w_a)
    wn0a = W_n0[0:hd]
    wn0b = W_n0[hd:]

    tp, tq = _tables_call(h, w_c, w_d, 1000)
    xflat = jnp.pad(x, ((0, 0), (0, 1))).reshape(-1)

    efp = edge_feat.reshape(e // 8, 128)
    g0 = _efhead_call(efp, w_til, 1000).reshape(e, 128)

    fd, fs, rxp, ryp, rzp, rp = _gather_call(tp, tq, dst3, src3, xflat, e)
    rp3 = rp[0:e // 128].reshape(e // bc, gpb, 128)

    gated, xw3 = _edge_call(
        fd, fs, g0, rp3, w_b, b_e0.reshape(1, -1), W_e1, b_e1.reshape(1, -1),
        W_inf, b_inf.reshape(1, 1), W_x0, b_x0.reshape(1, -1),
        W_x1, b_x1.reshape(1, 1), ng, r_coeff, r_step, bc)

    xwp = jnp.pad(xw3.reshape(e // 128, 128), ((0, rxp.shape[0] - e // 128),
                                               (0, 0)))

    zeros_nf = jnp.zeros((n, 128), jnp.float32)
    acc, xacc = _scatter_call(gated, dst3, xwp, rxp, ryp, rzp, zeros_nf, n, e)

    h_new = _node_call(acc, h, wn0a, wn0b, b_n0.reshape(1, -1),
                       W_n1, b_n1.reshape(1, -1), 1000)

    n_half = n // _NC
    dxf = jnp.concatenate([xacc[0, 0:4 * n_half], xacc[1, 0:4 * n_half]])
    xa = jnp.pad(dxf, (0, 960)).reshape(-1, 128)
    xpad = jnp.pad(x, ((0, 240), (0, 1))).reshape(-1, 128)
    xnp = _xfin_call(xpad, xa)
    x_new = xnp.reshape(-1, 4)[0:n, 0:3]
    return (h_new, x_new)


# speedup vs baseline: 4.9265x; 1.0786x over previous
import functools

import jax
import jax.numpy as jnp
from jax import lax
from jax.experimental import pallas as pl
from jax.experimental.pallas import tpu as pltpu
from jax.experimental.pallas import tpu_sc as plsc

_NC = 2
_NS = 16
_NW = _NC * _NS
_L = 16
_CH = 512
_SUB = 128


def _sc_mesh():
    return plsc.VectorSubcoreMesh(
        core_axis_name="c", subcore_axis_name="s", num_cores=_NC,
        num_subcores=_NS)


def _table_body(h_ref, w_ref, out_ref):
    out_ref[...] = jnp.dot(h_ref[...], w_ref[...],
                           preferred_element_type=jnp.float32)


def _table2_body(h_ref, wc_ref, wd_ref, p_ref, q_ref):
    h = h_ref[...]
    p_ref[...] = jnp.dot(h, wc_ref[...], preferred_element_type=jnp.float32)
    q_ref[...] = jnp.dot(h, wd_ref[...], preferred_element_type=jnp.float32)


def _tables_call(h, w_c, w_d, bn):
    n = h.shape[0]
    return pl.pallas_call(
        _table2_body,
        grid=(n // bn,),
        in_specs=[
            pl.BlockSpec((bn, h.shape[1]), lambda i: (i, 0)),
            pl.BlockSpec(w_c.shape, lambda i: (0, 0)),
            pl.BlockSpec(w_d.shape, lambda i: (0, 0)),
        ],
        out_specs=[
            pl.BlockSpec((bn, 128), lambda i: (i, 0)),
            pl.BlockSpec((bn, 128), lambda i: (i, 0)),
        ],
        out_shape=[
            jax.ShapeDtypeStruct((n, 128), jnp.float32),
            jax.ShapeDtypeStruct((n, 128), jnp.float32),
        ],
    )(h, w_c, w_d)


def _efhead_call(efp, w_til, br):
    r = efp.shape[0]
    return pl.pallas_call(
        _table_body,
        grid=(r // br,),
        in_specs=[
            pl.BlockSpec((br, 128), lambda i: (i, 0)),
            pl.BlockSpec(w_til.shape, lambda i: (0, 0)),
        ],
        out_specs=pl.BlockSpec((br, w_til.shape[1]), lambda i: (i, 0)),
        out_shape=jax.ShapeDtypeStruct((r, w_til.shape[1]), jnp.float32),
    )(efp, w_til)


def _gather_call(tp, tq, dst3, src3, xflat, e_total):
    grp = 1024
    n_grp = e_total // grp
    base_p, extra_p = divmod(n_grp, _NW)
    rrows = 8 * n_grp + 8
    pk = jax.ShapeDtypeStruct((rrows, 128), jnp.float32)

    @functools.partial(
        pl.kernel,
        out_type=(
            jax.ShapeDtypeStruct((e_total, 128), jnp.float32),
            pk, pk, pk, pk,
        ),
        mesh=_sc_mesh(),
        scratch_types=[
            pltpu.VMEM((8, 1, _SUB), jnp.int32),
            pltpu.VMEM((8, 1, _SUB), jnp.int32),
            pltpu.VMEM((128, 128), jnp.float32),
            pltpu.VMEM((128, 128), jnp.float32),
            pltpu.VMEM((128, 128), jnp.float32),
            pltpu.VMEM((128, 128), jnp.float32),
            pltpu.VMEM((4 * tp.shape[0],), jnp.float32),
            pltpu.VMEM((8, 128), jnp.float32),
            pltpu.VMEM((8, 128), jnp.float32),
            pltpu.VMEM((8, 128), jnp.float32),
            pltpu.VMEM((8, 128), jnp.float32),
            pltpu.SemaphoreType.DMA,
            pltpu.SemaphoreType.DMA,
            pltpu.SemaphoreType.DMA,
            pltpu.SemaphoreType.DMA,
            pltpu.SemaphoreType.DMA,
            pltpu.SemaphoreType.DMA,
            pltpu.SemaphoreType.DMA,
            pltpu.SemaphoreType.DMA,
        ],
        compiler_params=pltpu.CompilerParams(needs_layout_passes=False),
    )
    def k(tp_hbm, tq_hbm, dst_hbm, src_hbm, x_hbm,
          fd_hbm, rx_hbm, ry_hbm, rz_hbm, r_hbm,
          didx, sidx, pba, pbb, qba, qbb, xtab, rxb, ryb, rzb, rb,
          gp0, gp1, gq0, gq1, ws0, ws1, isem, rwsem):
        cid = lax.axis_index("c")
        sid = lax.axis_index("s")
        wid = sid * _NC + cid
        trips = jnp.where(wid < extra_p, base_p + 1, base_p)

        pltpu.sync_copy(x_hbm, xtab)
        pbufs = (pba, pbb)
        qbufs = (qba, qbb)
        gpsems = (gp0, gp1)
        gqsems = (gq0, gq1)
        wsems = (ws0, ws1)

        def fire_gather(g, s):
            par = s % 2
            return [pltpu.async_copy(tp_hbm.at[didx.at[s, 0]], pbufs[par],
                                     gpsems[par]),
                    pltpu.async_copy(tq_hbm.at[sidx.at[s, 0]], qbufs[par],
                                     gqsems[par])]

        def fold(s):
            par = s % 2
            pb, qb = pbufs[par], qbufs[par]

            def body(i, _):
                for v in range(8):
                    sl = pl.ds(v * _L, _L)
                    pb[i, sl] = pb[i, sl] + qb[i, sl]
                return 0
            lax.fori_loop(0, 128, body, 0, unroll=False)

        def fire_write(g, s):
            return pltpu.async_copy(
                pbufs[s % 2], fd_hbm.at[pl.ds(g * grp + s * 128, 128)],
                wsems[s % 2])

        def rel_compute(s):
            for v in range(8):
                l0 = v * _L
                d = didx[s, 0, pl.ds(l0, _L)]
                sx = sidx[s, 0, pl.ds(l0, _L)]
                relc = []
                for comp in range(3):
                    xd = plsc.load_gather(xtab, [d * 4 + comp])
                    xs = plsc.load_gather(xtab, [sx * 4 + comp])
                    relc.append(xd - xs)
                r2 = relc[0] * relc[0] + relc[1] * relc[1] + relc[2] * relc[2]
                rxb[s, pl.ds(l0, _L)] = relc[0]
                ryb[s, pl.ds(l0, _L)] = relc[1]
                rzb[s, pl.ds(l0, _L)] = relc[2]
                rb[s, pl.ds(l0, _L)] = r2

        def do_group(g, n_steps, pipelined):
            icps = [pltpu.async_copy(dst_hbm.at[pl.ds(8 * g, 8)], didx,
                                     isem),
                    pltpu.async_copy(src_hbm.at[pl.ds(8 * g, 8)], sidx,
                                     isem)]
            for cp in icps:
                cp.wait()
            gcps = {0: fire_gather(g, 0)}
            wcps = {}
            for s in range(n_steps):
                if s + 1 < n_steps:
                    if s - 1 >= 0:
                        wcps.pop(s - 1).wait()
                    gcps[s + 1] = fire_gather(g, s + 1)
                for cp in gcps.pop(s):
                    cp.wait()
                fold(s)
                wcps[s] = fire_write(g, s)
                rel_compute(s)
            for s in sorted(wcps):
                wcps[s].wait()
            if pipelined:
                pltpu.async_copy(rxb, rx_hbm.at[pl.ds(8 * g, 8)], rwsem)
                pltpu.async_copy(ryb, ry_hbm.at[pl.ds(8 * g, 8)], rwsem)
                pltpu.async_copy(rzb, rz_hbm.at[pl.ds(8 * g, 8)], rwsem)
                pltpu.async_copy(rb, r_hbm.at[pl.ds(8 * g, 8)], rwsem)
            else:
                pltpu.sync_copy(rxb, rx_hbm.at[pl.ds(8 * g, 8)])
                pltpu.sync_copy(ryb, ry_hbm.at[pl.ds(8 * g, 8)])
                pltpu.sync_copy(rzb, rz_hbm.at[pl.ds(8 * g, 8)])
                pltpu.sync_copy(rb, r_hbm.at[pl.ds(8 * g, 8)])

        def drain_rel():
            for _ in range(4):
                pltpu.make_async_copy(rxb, rx_hbm.at[pl.ds(0, 8)],
                                      rwsem).wait()

        def group(t, _):
            @pl.when(t > 0)
            def _():
                drain_rel()
            do_group(wid + t * _NW, 8, True)
            return 0

        lax.fori_loop(0, trips, group, 0, unroll=False)
        drain_rel()

        @pl.when(wid == 0)
        def _():
            pltpu.sync_copy(dst_hbm.at[pl.ds(8 * n_grp, 4)],
                            didx.at[pl.ds(0, 4)])
            pltpu.sync_copy(src_hbm.at[pl.ds(8 * n_grp, 4)],
                            sidx.at[pl.ds(0, 4)])
            gcps = {0: fire_gather(n_grp, 0)}
            wcps = {}
            for s in range(4):
                if s + 1 < 4:
                    if s - 1 >= 0:
                        wcps.pop(s - 1).wait()
                    gcps[s + 1] = fire_gather(n_grp, s + 1)
                for cp in gcps.pop(s):
                    cp.wait()
                fold(s)
                wcps[s] = fire_write(n_grp, s)
                rel_compute(s)
            for s in sorted(wcps):
                wcps[s].wait()
            pltpu.sync_copy(rxb, rx_hbm.at[pl.ds(8 * n_grp, 8)])
            pltpu.sync_copy(ryb, ry_hbm.at[pl.ds(8 * n_grp, 8)])
            pltpu.sync_copy(rzb, rz_hbm.at[pl.ds(8 * n_grp, 8)])
            pltpu.sync_copy(rb, r_hbm.at[pl.ds(8 * n_grp, 8)])

    return k(tp, tq, dst3, src3, xflat)


def _edge_body(ng, r_coeff, r_step, gpb, fd_ref, g0_ref, rp_ref,
               wb_ref, be0_ref, we1_ref, be1_ref, winf_ref, binf_ref,
               wx0_ref, bx0_ref, wx1_ref, bx1_ref, out_ref, xw_ref):
    bc = gpb * 128
    rpk = rp_ref[0]
    rt = jnp.transpose(rpk)
    r = jnp.concatenate([rt[:, g:g + 1] for g in range(gpb)], axis=0)
    offs = lax.broadcasted_iota(jnp.int32, (1, ng), 1).astype(jnp.float32)
    offs = offs * r_step
    r_feat = jnp.exp(r_coeff * (r - offs) ** 2)
    pre = (fd_ref[...] + g0_ref[...] + be0_ref[...]
           + jnp.dot(r_feat, wb_ref[...], preferred_element_type=jnp.float32))
    u = jnp.maximum(pre, 0.0)
    mij = jnp.maximum(
        jnp.dot(u, we1_ref[...], preferred_element_type=jnp.float32)
        + be1_ref[...], 0.0)
    z = jnp.dot(mij, winf_ref[...],
                preferred_element_type=jnp.float32) + binf_ref[...]
    eij = 1.0 / (1.0 + jnp.exp(-z))
    t = jnp.maximum(
        jnp.dot(mij, wx0_ref[...], preferred_element_type=jnp.float32)
        + bx0_ref[...], 0.0)
    xw = jnp.dot(t, wx1_ref[...],
                 preferred_element_type=jnp.float32) + bx1_ref[...]
    out_ref[...] = mij * eij
    xw_ref[...] = jnp.reshape(xw, (1, gpb, 128))


def _edge_call(fd, g0, rp3, wb, b_e0, w_e1, b_e1, w_inf, b_inf,
               w_x0, b_x0, w_x1, b_x1, ng, r_coeff, r_step, bc):
    e_total = fd.shape[0]
    gpb = bc // 128
    nb = e_total // bc
    full = lambda a: pl.BlockSpec(a.shape, lambda i: tuple(0 for _ in a.shape))
    return pl.pallas_call(
        functools.partial(_edge_body, ng, r_coeff, r_step, gpb),
        grid=(nb,),
        in_specs=[
            pl.BlockSpec((bc, 128), lambda i: (i, 0)),
            pl.BlockSpec((bc, 128), lambda i: (i, 0)),
            pl.BlockSpec((1, gpb, 128), lambda i: (i, 0, 0)),
            full(wb), full(b_e0), full(w_e1), full(b_e1), full(w_inf),
            full(b_inf), full(w_x0), full(b_x0), full(w_x1), full(b_x1),
        ],
        out_specs=[
            pl.BlockSpec((bc, 128), lambda i: (i, 0)),
            pl.BlockSpec((1, gpb, 128), lambda i: (i, 0, 0)),
        ],
        out_shape=[
            jax.ShapeDtypeStruct((e_total, 128), jnp.float32),
            jax.ShapeDtypeStruct((nb, gpb, 128), jnp.float32),
        ],
    )(fd, g0, rp3, wb, b_e0, w_e1, b_e1, w_inf, b_inf, w_x0, b_x0,
      w_x1, b_x1)


def _scatter_call(gated, dst3, xwp, rxp, ryp, rzp, zeros_nf, n_total,
                  e_total):
    n_grp = e_total // 1024
    base_p, extra_p = divmod(n_grp, _NS)
    n_half = n_total // _NC
    rpt = 312
    xacc_len = 1280 * _NS

    @functools.partial(
        pl.kernel,
        out_type=(
            jax.ShapeDtypeStruct((_NC, n_half, 128), jnp.float32),
            jax.ShapeDtypeStruct((_NC, xacc_len), jnp.float32),
        ),
        mesh=_sc_mesh(),
        scratch_types=[
            pltpu.VMEM((8, 1, _SUB), jnp.int32),
            pltpu.VMEM((8, 1, _SUB), jnp.int32),
            pltpu.VMEM((256, 128), jnp.float32),
            pltpu.VMEM((256, 128), jnp.float32),
            pltpu.VMEM((8, 128), jnp.float32),
            pltpu.VMEM((8, 128), jnp.float32),
            pltpu.VMEM((8, 128), jnp.float32),
            pltpu.VMEM((8, 128), jnp.float32),
            pltpu.VMEM((4096,), jnp.float32),
            pltpu.VMEM((32, 1, _SUB), jnp.int32),
            pltpu.VMEM((1280,), jnp.float32),
            pltpu.VMEM_SHARED((n_half, 128), jnp.float32),
            pltpu.VMEM_SHARED((xacc_len,), jnp.float32),
            pltpu.SemaphoreType.DMA,
            pltpu.SemaphoreType.DMA,
            pltpu.SemaphoreType.DMA,
            pltpu.SemaphoreType.DMA,
            pltpu.SemaphoreType.DMA,
        ],
        compiler_params=pltpu.CompilerParams(needs_layout_passes=False),
    )
    def k(g_hbm, dst_hbm, xw_hbm, rx_hbm, ry_hbm, rz_hbm, z_hbm,
          acc_hbm, xacc_hbm,
          didx, fidx, growa, growb, xwb, rxb, ryb, rzb, vals, idxs, zbuf,
          acc, xacc, ls0, ls1, as0, as1, esem):
        asems = (as0, as1)
        cid = lax.axis_index("c")
        sid = lax.axis_index("s")
        nbase = cid * n_half
        trips = jnp.where(sid < extra_p, base_p + 1, base_p)

        def zloop(i, _):
            zbuf[pl.ds(i * _L, _L)] = jnp.zeros((_L,), jnp.float32)
            return 0
        lax.fori_loop(0, 1280 // _L, zloop, 0, unroll=False)
        pltpu.sync_copy(zbuf, xacc.at[pl.ds(sid * 1280, 1280)])

        @pl.when(sid < _NS - 1)
        def _():
            pltpu.sync_copy(z_hbm.at[pl.ds(0, rpt)],
                            acc.at[pl.ds(sid * rpt, rpt)])

        @pl.when(sid == _NS - 1)
        def _():
            pltpu.sync_copy(z_hbm.at[pl.ds(0, 320)],
                            acc.at[pl.ds((_NS - 1) * rpt, 320)])

        plsc.subcore_barrier()

        def build_filtered(n_rows):
            for v in range(8 * n_rows):
                row = v // 8
                l0 = (v % 8) * _L
                d = didx[row, 0, pl.ds(l0, _L)] - nbase
                valid = (d >= 0) & (d < n_half)
                fidx[row, 0, pl.ds(l0, _L)] = jnp.where(valid, d, -1)
                xw = xwb[row, pl.ds(l0, _L)]
                base = v * 64
                pos0 = lax.iota(jnp.int32, _L) * 4
                for comp, rbuf in ((0, rxb), (1, ryb), (2, rzb)):
                    val = rbuf[row, pl.ds(l0, _L)] * xw
                    pos = pos0 + (base + comp)
                    plsc.store_scatter(vals, [pos], val)
                    plsc.store_scatter(
                        idxs, [pos // _SUB,
                               jnp.zeros((_L,), jnp.int32),
                               lax.rem(pos, _SUB)],
                        jnp.where(valid, d * 4 + comp, -1))
                pos = pos0 + (base + 3)
                plsc.store_scatter(
                    idxs, [pos // _SUB,
                           jnp.zeros((_L,), jnp.int32),
                           lax.rem(pos, _SUB)],
                    jnp.full((_L,), -1, jnp.int32))

        def do_group(g, n_sub):
            bufs = (growa, growb)
            lsems = (ls0, ls1)

            def fire_load(s):
                return pltpu.async_copy(
                    g_hbm.at[pl.ds(g * 1024 + s * 256, 256)],
                    bufs[s % 2], lsems[s % 2])

            hcps = [
                pltpu.async_copy(dst_hbm.at[pl.ds(8 * g, 2 * n_sub)],
                                 didx.at[pl.ds(0, 2 * n_sub)], esem),
                pltpu.async_copy(xw_hbm.at[pl.ds(8 * g, 8)], xwb, esem),
                pltpu.async_copy(rx_hbm.at[pl.ds(8 * g, 8)], rxb, esem),
                pltpu.async_copy(ry_hbm.at[pl.ds(8 * g, 8)], ryb, esem),
                pltpu.async_copy(rz_hbm.at[pl.ds(8 * g, 8)], rzb, esem),
            ]
            lcps = {0: fire_load(0)}
            for cp in hcps:
                cp.wait()
            build_filtered(2 * n_sub)
            ecps = [pltpu.async_copy(
                vals.at[pl.ds(t * _SUB, _SUB)],
                xacc.at[plsc.Indices(idxs.at[t, 0], ignored_value=-1)],
                esem, add=True) for t in range(8 * n_sub)]
            acps = {}
            for s in range(n_sub):
                if s + 1 < n_sub:
                    if s - 1 >= 0:
                        for cp in acps.pop(s - 1):
                            cp.wait()
                    lcps[s + 1] = fire_load(s + 1)
                lcps.pop(s).wait()
                acps[s] = [pltpu.async_copy(
                    bufs[s % 2].at[pl.ds(j * _SUB, _SUB)],
                    acc.at[plsc.Indices(fidx.at[2 * s + j, 0],
                                        ignored_value=-1)],
                    asems[s % 2], add=True) for j in range(2)]
            for s in sorted(acps):
                for cp in acps[s]:
                    cp.wait()
            for cp in ecps:
                cp.wait()

        def group(t, _):
            do_group(sid + t * _NS, 4)
            return 0

        lax.fori_loop(0, trips, group, 0, unroll=False)

        @pl.when(sid == 0)
        def _():
            do_group(n_grp, 2)

        plsc.subcore_barrier()

        @pl.when(sid < _NS - 1)
        def _():
            pltpu.sync_copy(acc.at[pl.ds(sid * rpt, rpt)],
                            acc_hbm.at[cid, pl.ds(sid * rpt, rpt)])

        @pl.when(sid == _NS - 1)
        def _():
            pltpu.sync_copy(acc.at[pl.ds((_NS - 1) * rpt, 320)],
                            acc_hbm.at[cid, pl.ds((_NS - 1) * rpt, 320)])

        pltpu.sync_copy(xacc.at[pl.ds(sid * 1280, 1280)],
                        xacc_hbm.at[cid, pl.ds(sid * 1280, 1280)])

    return k(gated, dst3, xwp, rxp, ryp, rzp, zeros_nf)


def _node_body(mi_ref, h_ref, wn0a_ref, wn0b_ref, bn0_ref,
               wn1_ref, bn1_ref, hout_ref):
    mi = mi_ref[0]
    h = h_ref[...]
    u = jnp.maximum(
        jnp.dot(mi, wn0a_ref[...], preferred_element_type=jnp.float32)
        + jnp.dot(h, wn0b_ref[...], preferred_element_type=jnp.float32)
        + bn0_ref[...], 0.0)
    hout_ref[...] = h + jnp.dot(
        u, wn1_ref[...], preferred_element_type=jnp.float32) + bn1_ref[...]


def _node_call(acc, h, wn0a, wn0b, b_n0, w_n1, b_n1, bn):
    n = h.shape[0]
    hpb = (n // _NC) // bn
    full = lambda a: pl.BlockSpec(a.shape, lambda i: tuple(0 for _ in a.shape))
    return pl.pallas_call(
        _node_body,
        grid=(n // bn,),
        in_specs=[
            pl.BlockSpec((1, bn, 128), lambda i: (i // hpb, i % hpb, 0)),
            pl.BlockSpec((bn, 128), lambda i: (i, 0)),
            full(wn0a), full(wn0b), full(b_n0), full(w_n1), full(b_n1),
        ],
        out_specs=pl.BlockSpec((bn, 128), lambda i: (i, 0)),
        out_shape=jax.ShapeDtypeStruct((n, 128), jnp.float32),
    )(acc, h, wn0a, wn0b, b_n0, w_n1, b_n1)


def _xfin_body(xp_ref, a0_ref, out_ref):
    out_ref[...] = xp_ref[...] + a0_ref[...]


def _xfin_call(xp, a0):
    spec = pl.BlockSpec(xp.shape, lambda: (0, 0))
    return pl.pallas_call(
        _xfin_body,
        in_specs=[spec, spec],
        out_specs=spec,
        out_shape=jax.ShapeDtypeStruct(xp.shape, jnp.float32),
    )(xp, a0)


def kernel(h, x, edge_feat, edge_index, W_e0, b_e0, W_e1, b_e1, W_inf, b_inf,
           W_x0, b_x0, W_x1, b_x1, W_n0, b_n0, W_n1, b_n1):
    n, hd = h.shape
    e = edge_index.shape[1]
    ef = edge_feat.shape[1]
    ng = W_e0.shape[0] - 2 * hd - ef
    r_step = 100.0 / (ng - 1)
    r_coeff = -0.5 / r_step ** 2
    bc = 2560
    gpb = bc // 128

    src = edge_index[0]
    dst = edge_index[1]
    dst3 = dst.reshape(e // _SUB, 1, _SUB)
    src3 = src.reshape(e // _SUB, 1, _SUB)

    w_a = W_e0[0:ef]
    w_b = W_e0[ef:ef + ng]
    w_c = W_e0[ef + ng:ef + ng + hd]
    w_d = W_e0[ef + ng + hd:]
    w_til = jnp.kron(jnp.eye(8, dtype=jnp.float32), w_a)
    wn0a = W_n0[0:hd]
    wn0b = W_n0[hd:]

    tp, tq = _tables_call(h, w_c, w_d, 1000)
    xflat = jnp.pad(x, ((0, 0), (0, 1))).reshape(-1)

    efp = edge_feat.reshape(e // 8, 128)
    g0 = _efhead_call(efp, w_til, 1000).reshape(e, 128)

    fd, rxp, ryp, rzp, rp = _gather_call(tp, tq, dst3, src3, xflat, e)
    rp3 = rp[0:e // 128].reshape(e // bc, gpb, 128)

    gated, xw3 = _edge_call(
        fd, g0, rp3, w_b, b_e0.reshape(1, -1), W_e1, b_e1.reshape(1, -1),
        W_inf, b_inf.reshape(1, 1), W_x0, b_x0.reshape(1, -1),
        W_x1, b_x1.reshape(1, 1), ng, r_coeff, r_step, bc)

    xwp = jnp.pad(xw3.reshape(e // 128, 128), ((0, rxp.shape[0] - e // 128),
                                               (0, 0)))

    zeros_nf = jnp.zeros((n, 128), jnp.float32)
    acc, xacc = _scatter_call(gated, dst3, xwp, rxp, ryp, rzp, zeros_nf, n, e)

    h_new = _node_call(acc, h, wn0a, wn0b, b_n0.reshape(1, -1),
                       W_n1, b_n1.reshape(1, -1), 1000)

    n_half = n // _NC
    dxf = jnp.concatenate([xacc[0, 0:4 * n_half], xacc[1, 0:4 * n_half]])
    xa = jnp.pad(dxf, (0, 960)).reshape(-1, 128)
    xpad = jnp.pad(x, ((0, 240), (0, 1))).reshape(-1, 128)
    xnp = _xfin_call(xpad, xa)
    x_new = xnp.reshape(-1, 4)[0:n, 0:3]
    return (h_new, x_new)


# speedup vs baseline: 5.0515x; 1.0254x over previous
import functools

import jax
import jax.numpy as jnp
from jax import lax
from jax.experimental import pallas as pl
from jax.experimental.pallas import tpu as pltpu
from jax.experimental.pallas import tpu_sc as plsc

_NC = 2
_NS = 16
_NW = _NC * _NS
_L = 16
_CH = 512
_SUB = 128


def _sc_mesh():
    return plsc.VectorSubcoreMesh(
        core_axis_name="c", subcore_axis_name="s", num_cores=_NC,
        num_subcores=_NS)


def _table_body(h_ref, w_ref, out_ref):
    out_ref[...] = jnp.dot(h_ref[...], w_ref[...],
                           preferred_element_type=jnp.float32)


def _prep_body(nh_blk, efp_ref, wt_ref, h_ref, wc_ref, wd_ref,
               g0_ref, p_ref, q_ref):
    g0_ref[...] = jnp.dot(efp_ref[...], wt_ref[...],
                          preferred_element_type=jnp.float32)
    h = h_ref[...]
    p_ref[...] = jnp.dot(h, wc_ref[...], preferred_element_type=jnp.float32)
    q_ref[...] = jnp.dot(h, wd_ref[...], preferred_element_type=jnp.float32)


def _prep_call(efp, w_til, h, w_c, w_d, br, bn):
    r = efp.shape[0]
    n = h.shape[0]
    nh_blk = n // bn
    return pl.pallas_call(
        functools.partial(_prep_body, nh_blk),
        grid=(r // br,),
        in_specs=[
            pl.BlockSpec((br, 128), lambda i: (i, 0)),
            pl.BlockSpec(w_til.shape, lambda i: (0, 0)),
            pl.BlockSpec((bn, h.shape[1]), lambda i: (i % nh_blk, 0)),
            pl.BlockSpec(w_c.shape, lambda i: (0, 0)),
            pl.BlockSpec(w_d.shape, lambda i: (0, 0)),
        ],
        out_specs=[
            pl.BlockSpec((br, w_til.shape[1]), lambda i: (i, 0)),
            pl.BlockSpec((bn, 128), lambda i: (i % nh_blk, 0)),
            pl.BlockSpec((bn, 128), lambda i: (i % nh_blk, 0)),
        ],
        out_shape=[
            jax.ShapeDtypeStruct((r, w_til.shape[1]), jnp.float32),
            jax.ShapeDtypeStruct((n, 128), jnp.float32),
            jax.ShapeDtypeStruct((n, 128), jnp.float32),
        ],
    )(efp, w_til, h, w_c, w_d)


def _gather_call(tp, tq, dst3, src3, xflat, e_total):
    grp = 1024
    n_grp = e_total // grp
    base_p, extra_p = divmod(n_grp, _NW)
    rrows = 8 * n_grp + 8
    pk = jax.ShapeDtypeStruct((rrows, 128), jnp.float32)

    @functools.partial(
        pl.kernel,
        out_type=(
            jax.ShapeDtypeStruct((e_total, 128), jnp.float32),
            pk, pk, pk, pk,
        ),
        mesh=_sc_mesh(),
        scratch_types=[
            pltpu.VMEM((8, 1, _SUB), jnp.int32),
            pltpu.VMEM((8, 1, _SUB), jnp.int32),
            pltpu.VMEM((128, 128), jnp.float32),
            pltpu.VMEM((128, 128), jnp.float32),
            pltpu.VMEM((128, 128), jnp.float32),
            pltpu.VMEM((128, 128), jnp.float32),
            pltpu.VMEM((4 * tp.shape[0],), jnp.float32),
            pltpu.VMEM((8, 128), jnp.float32),
            pltpu.VMEM((8, 128), jnp.float32),
            pltpu.VMEM((8, 128), jnp.float32),
            pltpu.VMEM((8, 128), jnp.float32),
            pltpu.SemaphoreType.DMA,
            pltpu.SemaphoreType.DMA,
            pltpu.SemaphoreType.DMA,
            pltpu.SemaphoreType.DMA,
            pltpu.SemaphoreType.DMA,
            pltpu.SemaphoreType.DMA,
            pltpu.SemaphoreType.DMA,
            pltpu.SemaphoreType.DMA,
        ],
        compiler_params=pltpu.CompilerParams(needs_layout_passes=False),
    )
    def k(tp_hbm, tq_hbm, dst_hbm, src_hbm, x_hbm,
          fd_hbm, rx_hbm, ry_hbm, rz_hbm, r_hbm,
          didx, sidx, pba, pbb, qba, qbb, xtab, rxb, ryb, rzb, rb,
          gp0, gp1, gq0, gq1, ws0, ws1, isem, rwsem):
        cid = lax.axis_index("c")
        sid = lax.axis_index("s")
        wid = sid * _NC + cid
        trips = jnp.where(wid < extra_p, base_p + 1, base_p)

        pltpu.sync_copy(x_hbm, xtab)
        pbufs = (pba, pbb)
        qbufs = (qba, qbb)
        gpsems = (gp0, gp1)
        gqsems = (gq0, gq1)
        wsems = (ws0, ws1)

        def fire_gather(g, s):
            par = s % 2
            return [pltpu.async_copy(tp_hbm.at[didx.at[s, 0]], pbufs[par],
                                     gpsems[par]),
                    pltpu.async_copy(tq_hbm.at[sidx.at[s, 0]], qbufs[par],
                                     gqsems[par])]

        def fold(s):
            par = s % 2
            pb, qb = pbufs[par], qbufs[par]

            def body(i, _):
                for v in range(8):
                    sl = pl.ds(v * _L, _L)
                    pb[i, sl] = pb[i, sl] + qb[i, sl]
                return 0
            lax.fori_loop(0, 128, body, 0, unroll=False)

        def fire_write(g, s):
            return pltpu.async_copy(
                pbufs[s % 2], fd_hbm.at[pl.ds(g * grp + s * 128, 128)],
                wsems[s % 2])

        def rel_compute(s):
            for v in range(8):
                l0 = v * _L
                d = didx[s, 0, pl.ds(l0, _L)]
                sx = sidx[s, 0, pl.ds(l0, _L)]
                relc = []
                for comp in range(3):
                    xd = plsc.load_gather(xtab, [d * 4 + comp])
                    xs = plsc.load_gather(xtab, [sx * 4 + comp])
                    relc.append(xd - xs)
                r2 = relc[0] * relc[0] + relc[1] * relc[1] + relc[2] * relc[2]
                rxb[s, pl.ds(l0, _L)] = relc[0]
                ryb[s, pl.ds(l0, _L)] = relc[1]
                rzb[s, pl.ds(l0, _L)] = relc[2]
                rb[s, pl.ds(l0, _L)] = r2

        def do_group(g, n_steps, pipelined):
            icps = [pltpu.async_copy(dst_hbm.at[pl.ds(8 * g, 8)], didx,
                                     isem),
                    pltpu.async_copy(src_hbm.at[pl.ds(8 * g, 8)], sidx,
                                     isem)]
            for cp in icps:
                cp.wait()
            gcps = {0: fire_gather(g, 0)}
            wcps = {}
            for s in range(n_steps):
                if s + 1 < n_steps:
                    if s - 1 >= 0:
                        wcps.pop(s - 1).wait()
                    gcps[s + 1] = fire_gather(g, s + 1)
                for cp in gcps.pop(s):
                    cp.wait()
                fold(s)
                wcps[s] = fire_write(g, s)
                rel_compute(s)
            for s in sorted(wcps):
                wcps[s].wait()
            if pipelined:
                pltpu.async_copy(rxb, rx_hbm.at[pl.ds(8 * g, 8)], rwsem)
                pltpu.async_copy(ryb, ry_hbm.at[pl.ds(8 * g, 8)], rwsem)
                pltpu.async_copy(rzb, rz_hbm.at[pl.ds(8 * g, 8)], rwsem)
                pltpu.async_copy(rb, r_hbm.at[pl.ds(8 * g, 8)], rwsem)
            else:
                pltpu.sync_copy(rxb, rx_hbm.at[pl.ds(8 * g, 8)])
                pltpu.sync_copy(ryb, ry_hbm.at[pl.ds(8 * g, 8)])
                pltpu.sync_copy(rzb, rz_hbm.at[pl.ds(8 * g, 8)])
                pltpu.sync_copy(rb, r_hbm.at[pl.ds(8 * g, 8)])

        def drain_rel():
            for _ in range(4):
                pltpu.make_async_copy(rxb, rx_hbm.at[pl.ds(0, 8)],
                                      rwsem).wait()

        def group(t, _):
            @pl.when(t > 0)
            def _():
                drain_rel()
            do_group(wid + t * _NW, 8, True)
            return 0

        lax.fori_loop(0, trips, group, 0, unroll=False)
        drain_rel()

        @pl.when(wid == 0)
        def _():
            pltpu.sync_copy(dst_hbm.at[pl.ds(8 * n_grp, 4)],
                            didx.at[pl.ds(0, 4)])
            pltpu.sync_copy(src_hbm.at[pl.ds(8 * n_grp, 4)],
                            sidx.at[pl.ds(0, 4)])
            gcps = {0: fire_gather(n_grp, 0)}
            wcps = {}
            for s in range(4):
                if s + 1 < 4:
                    if s - 1 >= 0:
                        wcps.pop(s - 1).wait()
                    gcps[s + 1] = fire_gather(n_grp, s + 1)
                for cp in gcps.pop(s):
                    cp.wait()
                fold(s)
                wcps[s] = fire_write(n_grp, s)
                rel_compute(s)
            for s in sorted(wcps):
                wcps[s].wait()
            pltpu.sync_copy(rxb, rx_hbm.at[pl.ds(8 * n_grp, 8)])
            pltpu.sync_copy(ryb, ry_hbm.at[pl.ds(8 * n_grp, 8)])
            pltpu.sync_copy(rzb, rz_hbm.at[pl.ds(8 * n_grp, 8)])
            pltpu.sync_copy(rb, r_hbm.at[pl.ds(8 * n_grp, 8)])

    return k(tp, tq, dst3, src3, xflat)


def _edge_body(ng, r_coeff, r_step, gpb, fd_ref, g0_ref, rp_ref,
               wb_ref, be0_ref, we1_ref, be1_ref, winf_ref, binf_ref,
               wx0_ref, bx0_ref, wx1_ref, bx1_ref, out_ref, xw_ref):
    bc = gpb * 128
    rpk = rp_ref[0]
    rt = jnp.transpose(rpk)
    r = jnp.concatenate([rt[:, g:g + 1] for g in range(gpb)], axis=0)
    offs = lax.broadcasted_iota(jnp.int32, (1, ng), 1).astype(jnp.float32)
    offs = offs * r_step
    r_feat = jnp.exp(r_coeff * (r - offs) ** 2)
    pre = (fd_ref[...] + g0_ref[...] + be0_ref[...]
           + jnp.dot(r_feat, wb_ref[...], preferred_element_type=jnp.float32))
    u = jnp.maximum(pre, 0.0)
    mij = jnp.maximum(
        jnp.dot(u, we1_ref[...], preferred_element_type=jnp.float32)
        + be1_ref[...], 0.0)
    z = jnp.dot(mij, winf_ref[...],
                preferred_element_type=jnp.float32) + binf_ref[...]
    eij = 1.0 / (1.0 + jnp.exp(-z))
    t = jnp.maximum(
        jnp.dot(mij, wx0_ref[...], preferred_element_type=jnp.float32)
        + bx0_ref[...], 0.0)
    xw = jnp.dot(t, wx1_ref[...],
                 preferred_element_type=jnp.float32) + bx1_ref[...]
    out_ref[...] = mij * eij
    xw_ref[...] = jnp.reshape(xw, (1, gpb, 128))


def _edge_call(fd, g0, rp3, wb, b_e0, w_e1, b_e1, w_inf, b_inf,
               w_x0, b_x0, w_x1, b_x1, ng, r_coeff, r_step, bc):
    e_total = fd.shape[0]
    gpb = bc // 128
    nb = e_total // bc
    full = lambda a: pl.BlockSpec(a.shape, lambda i: tuple(0 for _ in a.shape))
    return pl.pallas_call(
        functools.partial(_edge_body, ng, r_coeff, r_step, gpb),
        grid=(nb,),
        in_specs=[
            pl.BlockSpec((bc, 128), lambda i: (i, 0)),
            pl.BlockSpec((bc, 128), lambda i: (i, 0)),
            pl.BlockSpec((1, gpb, 128), lambda i: (i, 0, 0)),
            full(wb), full(b_e0), full(w_e1), full(b_e1), full(w_inf),
            full(b_inf), full(w_x0), full(b_x0), full(w_x1), full(b_x1),
        ],
        out_specs=[
            pl.BlockSpec((bc, 128), lambda i: (i, 0)),
            pl.BlockSpec((1, gpb, 128), lambda i: (i, 0, 0)),
        ],
        out_shape=[
            jax.ShapeDtypeStruct((e_total, 128), jnp.float32),
            jax.ShapeDtypeStruct((nb, gpb, 128), jnp.float32),
        ],
    )(fd, g0, rp3, wb, b_e0, w_e1, b_e1, w_inf, b_inf, w_x0, b_x0,
      w_x1, b_x1)


def _scatter_call(gated, dst3, xwp, rxp, ryp, rzp, zeros_nf, n_total,
                  e_total):
    n_grp = e_total // 1024
    base_p, extra_p = divmod(n_grp, _NS)
    n_half = n_total // _NC
    rpt = 312
    xacc_len = 1280 * _NS

    @functools.partial(
        pl.kernel,
        out_type=(
            jax.ShapeDtypeStruct((_NC, n_half, 128), jnp.float32),
            jax.ShapeDtypeStruct((_NC, xacc_len), jnp.float32),
        ),
        mesh=_sc_mesh(),
        scratch_types=[
            pltpu.VMEM((8, 1, _SUB), jnp.int32),
            pltpu.VMEM((8, 1, _SUB), jnp.int32),
            pltpu.VMEM((256, 128), jnp.float32),
            pltpu.VMEM((256, 128), jnp.float32),
            pltpu.VMEM((8, 128), jnp.float32),
            pltpu.VMEM((8, 128), jnp.float32),
            pltpu.VMEM((8, 128), jnp.float32),
            pltpu.VMEM((8, 128), jnp.float32),
            pltpu.VMEM((4096,), jnp.float32),
            pltpu.VMEM((32, 1, _SUB), jnp.int32),
            pltpu.VMEM((1280,), jnp.float32),
            pltpu.VMEM_SHARED((n_half, 128), jnp.float32),
            pltpu.VMEM_SHARED((xacc_len,), jnp.float32),
            pltpu.SemaphoreType.DMA,
            pltpu.SemaphoreType.DMA,
            pltpu.SemaphoreType.DMA,
            pltpu.SemaphoreType.DMA,
            pltpu.SemaphoreType.DMA,
        ],
        compiler_params=pltpu.CompilerParams(needs_layout_passes=False),
    )
    def k(g_hbm, dst_hbm, xw_hbm, rx_hbm, ry_hbm, rz_hbm, z_hbm,
          acc_hbm, xacc_hbm,
          didx, fidx, growa, growb, xwb, rxb, ryb, rzb, vals, idxs, zbuf,
          acc, xacc, ls0, ls1, as0, as1, esem):
        asems = (as0, as1)
        cid = lax.axis_index("c")
        sid = lax.axis_index("s")
        nbase = cid * n_half
        trips = jnp.where(sid < extra_p, base_p + 1, base_p)

        def zloop(i, _):
            zbuf[pl.ds(i * _L, _L)] = jnp.zeros((_L,), jnp.float32)
            return 0
        lax.fori_loop(0, 1280 // _L, zloop, 0, unroll=False)
        pltpu.sync_copy(zbuf, xacc.at[pl.ds(sid * 1280, 1280)])

        @pl.when(sid < _NS - 1)
        def _():
            pltpu.sync_copy(z_hbm.at[pl.ds(0, rpt)],
                            acc.at[pl.ds(sid * rpt, rpt)])

        @pl.when(sid == _NS - 1)
        def _():
            pltpu.sync_copy(z_hbm.at[pl.ds(0, 320)],
                            acc.at[pl.ds((_NS - 1) * rpt, 320)])

        plsc.subcore_barrier()

        def build_filtered(n_rows):
            for v in range(8 * n_rows):
                row = v // 8
                l0 = (v % 8) * _L
                d = didx[row, 0, pl.ds(l0, _L)] - nbase
                valid = (d >= 0) & (d < n_half)
                fidx[row, 0, pl.ds(l0, _L)] = jnp.where(valid, d, -1)
                xw = xwb[row, pl.ds(l0, _L)]
                base = v * 64
                pos0 = lax.iota(jnp.int32, _L) * 4
                for comp, rbuf in ((0, rxb), (1, ryb), (2, rzb)):
                    val = rbuf[row, pl.ds(l0, _L)] * xw
                    pos = pos0 + (base + comp)
                    plsc.store_scatter(vals, [pos], val)
                    plsc.store_scatter(
                        idxs, [pos // _SUB,
                               jnp.zeros((_L,), jnp.int32),
                               lax.rem(pos, _SUB)],
                        jnp.where(valid, d * 4 + comp, -1))
                pos = pos0 + (base + 3)
                plsc.store_scatter(
                    idxs, [pos // _SUB,
                           jnp.zeros((_L,), jnp.int32),
                           lax.rem(pos, _SUB)],
                    jnp.full((_L,), -1, jnp.int32))

        def do_group(g, n_sub):
            bufs = (growa, growb)
            lsems = (ls0, ls1)

            def fire_load(s):
                return pltpu.async_copy(
                    g_hbm.at[pl.ds(g * 1024 + s * 256, 256)],
                    bufs[s % 2], lsems[s % 2])

            hcps = [
                pltpu.async_copy(dst_hbm.at[pl.ds(8 * g, 2 * n_sub)],
                                 didx.at[pl.ds(0, 2 * n_sub)], esem),
                pltpu.async_copy(xw_hbm.at[pl.ds(8 * g, 8)], xwb, esem),
                pltpu.async_copy(rx_hbm.at[pl.ds(8 * g, 8)], rxb, esem),
                pltpu.async_copy(ry_hbm.at[pl.ds(8 * g, 8)], ryb, esem),
                pltpu.async_copy(rz_hbm.at[pl.ds(8 * g, 8)], rzb, esem),
            ]
            lcps = {0: fire_load(0)}
            for cp in hcps:
                cp.wait()
            build_filtered(2 * n_sub)
            ecps = [pltpu.async_copy(
                vals.at[pl.ds(t * _SUB, _SUB)],
                xacc.at[plsc.Indices(idxs.at[t, 0], ignored_value=-1)],
                esem, add=True) for t in range(8 * n_sub)]
            acps = {}
            for s in range(n_sub):
                if s + 1 < n_sub:
                    if s - 1 >= 0:
                        for cp in acps.pop(s - 1):
                            cp.wait()
                    lcps[s + 1] = fire_load(s + 1)
                lcps.pop(s).wait()
                acps[s] = [pltpu.async_copy(
                    bufs[s % 2].at[pl.ds(j * _SUB, _SUB)],
                    acc.at[plsc.Indices(fidx.at[2 * s + j, 0],
                                        ignored_value=-1)],
                    asems[s % 2], add=True) for j in range(2)]
            for s in sorted(acps):
                for cp in acps[s]:
                    cp.wait()
            for cp in ecps:
                cp.wait()

        def group(t, _):
            do_group(sid + t * _NS, 4)
            return 0

        lax.fori_loop(0, trips, group, 0, unroll=False)

        @pl.when(sid == 0)
        def _():
            do_group(n_grp, 2)

        plsc.subcore_barrier()

        @pl.when(sid < _NS - 1)
        def _():
            pltpu.sync_copy(acc.at[pl.ds(sid * rpt, rpt)],
                            acc_hbm.at[cid, pl.ds(sid * rpt, rpt)])

        @pl.when(sid == _NS - 1)
        def _():
            pltpu.sync_copy(acc.at[pl.ds((_NS - 1) * rpt, 320)],
                            acc_hbm.at[cid, pl.ds((_NS - 1) * rpt, 320)])

        pltpu.sync_copy(xacc.at[pl.ds(sid * 1280, 1280)],
                        xacc_hbm.at[cid, pl.ds(sid * 1280, 1280)])

    return k(gated, dst3, xwp, rxp, ryp, rzp, zeros_nf)


def _node_body(mi_ref, h_ref, wn0a_ref, wn0b_ref, bn0_ref,
               wn1_ref, bn1_ref, xp_ref, xa_ref, hout_ref, xout_ref):
    mi = mi_ref[0]
    h = h_ref[...]
    u = jnp.maximum(
        jnp.dot(mi, wn0a_ref[...], preferred_element_type=jnp.float32)
        + jnp.dot(h, wn0b_ref[...], preferred_element_type=jnp.float32)
        + bn0_ref[...], 0.0)
    hout_ref[...] = h + jnp.dot(
        u, wn1_ref[...], preferred_element_type=jnp.float32) + bn1_ref[...]

    @pl.when(pl.program_id(0) == 0)
    def _():
        xout_ref[...] = xp_ref[...] + xa_ref[...]


def _node_call(acc, h, wn0a, wn0b, b_n0, w_n1, b_n1, xp, xa, bn):
    n = h.shape[0]
    hpb = (n // _NC) // bn
    full = lambda a: pl.BlockSpec(a.shape, lambda i: tuple(0 for _ in a.shape))
    return pl.pallas_call(
        _node_body,
        grid=(n // bn,),
        in_specs=[
            pl.BlockSpec((1, bn, 128), lambda i: (i // hpb, i % hpb, 0)),
            pl.BlockSpec((bn, 128), lambda i: (i, 0)),
            full(wn0a), full(wn0b), full(b_n0), full(w_n1), full(b_n1),
            full(xp), full(xa),
        ],
        out_specs=[
            pl.BlockSpec((bn, 128), lambda i: (i, 0)),
            full(xp),
        ],
        out_shape=[
            jax.ShapeDtypeStruct((n, 128), jnp.float32),
            jax.ShapeDtypeStruct(xp.shape, jnp.float32),
        ],
    )(acc, h, wn0a, wn0b, b_n0, w_n1, b_n1, xp, xa)


def kernel(h, x, edge_feat, edge_index, W_e0, b_e0, W_e1, b_e1, W_inf, b_inf,
           W_x0, b_x0, W_x1, b_x1, W_n0, b_n0, W_n1, b_n1):
    n, hd = h.shape
    e = edge_index.shape[1]
    ef = edge_feat.shape[1]
    ng = W_e0.shape[0] - 2 * hd - ef
    r_step = 100.0 / (ng - 1)
    r_coeff = -0.5 / r_step ** 2
    bc = 6400
    gpb = bc // 128

    src = edge_index[0]
    dst = edge_index[1]
    dst3 = dst.reshape(e // _SUB, 1, _SUB)
    src3 = src.reshape(e // _SUB, 1, _SUB)

    w_a = W_e0[0:ef]
    w_b = W_e0[ef:ef + ng]
    w_c = W_e0[ef + ng:ef + ng + hd]
    w_d = W_e0[ef + ng + hd:]
    w_til = jnp.kron(jnp.eye(8, dtype=jnp.float32), w_a)
    wn0a = W_n0[0:hd]
    wn0b = W_n0[hd:]

    xflat = jnp.pad(x, ((0, 0), (0, 1))).reshape(-1)

    efp = edge_feat.reshape(e // 8, 128)
    g0w, tp, tq = _prep_call(efp, w_til, h, w_c, w_d, 1000, 1000)
    g0 = g0w.reshape(e, 128)

    fd, rxp, ryp, rzp, rp = _gather_call(tp, tq, dst3, src3, xflat, e)
    rp3 = rp[0:e // 128].reshape(e // bc, gpb, 128)

    gated, xw3 = _edge_call(
        fd, g0, rp3, w_b, b_e0.reshape(1, -1), W_e1, b_e1.reshape(1, -1),
        W_inf, b_inf.reshape(1, 1), W_x0, b_x0.reshape(1, -1),
        W_x1, b_x1.reshape(1, 1), ng, r_coeff, r_step, bc)

    xwp = jnp.pad(xw3.reshape(e // 128, 128), ((0, rxp.shape[0] - e // 128),
                                               (0, 0)))

    zeros_nf = jnp.zeros((n, 128), jnp.float32)
    acc, xacc = _scatter_call(gated, dst3, xwp, rxp, ryp, rzp, zeros_nf, n, e)

    n_half = n // _NC
    dxf = jnp.concatenate([xacc[0, 0:4 * n_half], xacc[1, 0:4 * n_half]])
    xa = jnp.pad(dxf, (0, 960)).reshape(-1, 128)
    xpad = jnp.pad(x, ((0, 240), (0, 1))).reshape(-1, 128)
    h_new, xnp = _node_call(acc, h, wn0a, wn0b, b_n0.reshape(1, -1),
                            W_n1, b_n1.reshape(1, -1), xpad, xa, 1000)
    x_new = xnp.reshape(-1, 4)[0:n, 0:3]
    return (h_new, x_new)
